# baseline (speedup 1.0000x reference)
import functools

import jax
import jax.numpy as jnp
from jax import lax
from jax.experimental import pallas as pl
from jax.experimental.pallas import tpu as pltpu
from jax.experimental.pallas import tpu_sc as plsc

G = 64
N_PER = 512
N = G * N_PER
M = 8192
NP_ = 8192
MP = 4096
K_SORT = 30

NC = 2
NS = 16
NW = NC * NS
CH = 128
NNZ1 = 524288
NNZ2 = 131072


_SPMEM_BUDGET = 1966080


@functools.lru_cache(maxsize=None)
def _make_spmm(nnz, nrows, d, src_rows, fused=False):
    epw = nnz // NW
    nch = epw // CH
    rpw = nrows // NS
    spw = src_rows // NS
    comb = 3 * CH * d if fused else 0
    sh_extra = src_rows * d if fused else 0
    assert not fused or (d % 16 == 0 and spw % CH == 0)
    K = 0
    NB = nch
    for cand in (8, 4, 2):
        if nch % cand == 0 and (
                NS * (cand * CH * d + 2 * nch * CH + comb)
                + nrows * d + sh_extra <= _SPMEM_BUDGET):
            K = cand
            break
    if K < 8 and nch > 16:
        for cand in (8, 4, 3, 2):
            if (NS * (cand * CH * d + 4 * 16 * CH + comb)
                    + nrows * d + sh_extra <= _SPMEM_BUDGET):
                if cand > K:
                    K = cand
                    NB = 16
                break
    assert K >= 2 and nch % NB == 0, (nnz, nrows, d, fused)
    nblk = nch // NB
    NLANE = d // 16
    mesh = plsc.VectorSubcoreMesh(
        core_axis_name="c", subcore_axis_name="s", num_cores=NC,
        num_subcores=NS)

    GA = K // 2
    LG = K - GA

    def body(*refs):
        if fused:
            (x_hbm, rdeg_hbm, gidx_hbm, sidx_hbm, zeros_hbm, out_hbm,
             gidx_v, sidx_v, rows_v, cb_v, acc_sh, src_sh, gsem,
             ssem) = refs
        else:
            (x_hbm, gidx_hbm, sidx_hbm, zeros_hbm, out_hbm,
             gidx_v, sidx_v, rows_v, acc_sh, gsem, ssem) = refs
            src_sh = x_hbm
        c = lax.axis_index("c")
        s = lax.axis_index("s")
        w = c * NS + s
        r0 = s * rpw
        pltpu.sync_copy(zeros_hbm.at[pl.ds(r0, rpw), :],
                        acc_sh.at[pl.ds(r0, rpw), :])
        if fused:
            for t in range(spw // CH):
                rs = s * spw + t * CH
                pltpu.sync_copy(x_hbm.at[0, pl.ds(rs, CH), :], cb_v.at[0])
                pltpu.sync_copy(x_hbm.at[1, pl.ds(rs, CH), :], cb_v.at[1])
                pltpu.sync_copy(rdeg_hbm.at[pl.ds(rs, CH), :], cb_v.at[2])

                def vstep(i, carry):
                    r = i // NLANE
                    o = (i % NLANE) * 16
                    cb_v[0, r, pl.ds(o, 16)] = (
                        (cb_v[0, r, pl.ds(o, 16)]
                         + cb_v[1, r, pl.ds(o, 16)])
                        * cb_v[2, r, pl.ds(o, 16)])
                    return carry

                lax.fori_loop(0, CH * NLANE, vstep, 0)
                pltpu.sync_copy(cb_v.at[0], src_sh.at[pl.ds(rs, CH), :])
        base = w * nch

        def stage_idx(b, buf):
            pltpu.sync_copy(gidx_hbm.at[pl.ds(base + b * NB, NB), :],
                            gidx_v.at[buf])
            pltpu.sync_copy(sidx_hbm.at[pl.ds(base + b * NB, NB), :],
                            sidx_v.at[buf])

        stage_idx(0, 0)
        plsc.subcore_barrier()

        def gather(j):
            pltpu.async_copy(
                src_sh.at[gidx_v.at[(j // NB) % 2, j % NB]],
                rows_v.at[pl.ds((j % K) * CH, CH), :], gsem)

        def gather_wait(j):
            pltpu.make_async_copy(
                src_sh.at[gidx_v.at[0, 0]],
                rows_v.at[pl.ds((j % K) * CH, CH), :], gsem).wait()

        def scat(j):
            pltpu.async_copy(
                rows_v.at[pl.ds((j % K) * CH, CH), :],
                acc_sh.at[sidx_v.at[(j // NB) % 2, j % NB]], ssem, add=True)

        def scat_wait(j):
            pltpu.make_async_copy(
                rows_v.at[pl.ds((j % K) * CH, CH), :],
                acc_sh.at[sidx_v.at[0, 0]], ssem).wait()

        for j in range(GA):
            gather(j)

        def blk(b, carry):
            @pl.when(b + 1 < nblk)
            def _():
                stage_idx(b + 1, (b + 1) % 2)

            def step(q, carry2):
                j = b * NB + q

                @pl.when(q >= LG)
                def _():
                    scat_wait(j - LG)

                @pl.when(j + GA < nch)
                def _():
                    gather(j + GA)
                gather_wait(j)
                scat(j)
                return carry2

            lax.fori_loop(0, NB, step, carry)
            def drain(q, carry2):
                scat_wait(b * NB + NB - LG + q)
                return carry2

            lax.fori_loop(0, LG, drain, carry)
            return carry

        lax.fori_loop(0, nblk, blk, 0)
        plsc.subcore_barrier()
        pltpu.sync_copy(acc_sh.at[pl.ds(r0, rpw), :],
                        out_hbm.at[c, pl.ds(r0, rpw), :])

    scratch = [
        pltpu.VMEM((2, NB, CH), jnp.int32) if nblk > 1
        else pltpu.VMEM((1, NB, CH), jnp.int32),
        pltpu.VMEM((2, NB, CH), jnp.int32) if nblk > 1
        else pltpu.VMEM((1, NB, CH), jnp.int32),
        pltpu.VMEM((K * CH, d), jnp.float32),
    ]
    if fused:
        scratch.append(pltpu.VMEM((3, CH, d), jnp.float32))
    scratch.append(pltpu.VMEM_SHARED((nrows, d), jnp.float32))
    if fused:
        scratch.append(pltpu.VMEM_SHARED((src_rows, d), jnp.float32))
    scratch += [pltpu.SemaphoreType.DMA, pltpu.SemaphoreType.DMA]
    return pl.kernel(
        body,
        out_type=jax.ShapeDtypeStruct((NC, nrows, d), jnp.float32),
        mesh=mesh,
        scratch_types=scratch,
        compiler_params=pltpu.CompilerParams(use_tc_tiling_on_sc=False),
        name="sc_spmm%s_%d_%d_%d" % ("f" if fused else "", nnz, nrows, d),
    )


@functools.lru_cache(maxsize=None)
def _make_degs():
    nch1 = (NNZ1 // NW) // CH
    nch2 = (NNZ2 // NW) // CH
    sizes = (N, M, NP_, MP)
    mesh = plsc.VectorSubcoreMesh(
        core_axis_name="c", subcore_axis_name="s", num_cores=NC,
        num_subcores=NS)

    def body(rn_hbm, cn_hbm, rp_hbm, cp_hbm, ones_hbm, zeros_hbm,
             on_hbm, om_hbm, onp_hbm, omp_hbm,
             rn_v, cn_v, rp_v, cp_v, ones_v, an, am, anp, amp, sem):
        c = lax.axis_index("c")
        s = lax.axis_index("s")
        w = c * NS + s
        accs = (an, am, anp, amp)
        outs = (on_hbm, om_hbm, onp_hbm, omp_hbm)
        for acc, r in zip(accs, sizes):
            rp = r // NS
            pltpu.sync_copy(zeros_hbm.at[pl.ds(0, rp), :],
                            acc.at[pl.ds(s * rp, rp), :])
        pltpu.sync_copy(ones_hbm, ones_v)
        pltpu.sync_copy(rn_hbm.at[pl.ds(w * nch1, nch1), :], rn_v)
        pltpu.sync_copy(cn_hbm.at[pl.ds(w * nch1, nch1), :], cn_v)
        pltpu.sync_copy(rp_hbm.at[pl.ds(w * nch2, nch2), :], rp_v)
        pltpu.sync_copy(cp_hbm.at[pl.ds(w * nch2, nch2), :], cp_v)
        plsc.subcore_barrier()

        for idx_v, nchl, acc in ((rn_v, nch1, an), (cn_v, nch1, am),
                                 (rp_v, nch2, anp), (cp_v, nch2, amp)):
            def st(j, carry, idx_v=idx_v, acc=acc):
                pltpu.async_copy(ones_v, acc.at[idx_v.at[j]], sem, add=True)

                @pl.when(j >= 8)
                def _():
                    pltpu.make_async_copy(
                        ones_v, acc.at[idx_v.at[0]], sem).wait()
                return carry

            lax.fori_loop(0, nchl, st, 0)
            for _ in range(min(8, nchl)):
                pltpu.make_async_copy(ones_v, acc.at[idx_v.at[0]],
                                      sem).wait()
        plsc.subcore_barrier()
        for acc, r, out in zip(accs, sizes, outs):
            rp = r // NS
            pltpu.sync_copy(acc.at[pl.ds(s * rp, rp), :],
                            out.at[c, pl.ds(s * rp, rp), :])

    return pl.kernel(
        body,
        out_type=[jax.ShapeDtypeStruct((NC, r, 8), jnp.float32)
                  for r in sizes],
        mesh=mesh,
        scratch_types=[
            pltpu.VMEM((nch1, CH), jnp.int32),
            pltpu.VMEM((nch1, CH), jnp.int32),
            pltpu.VMEM((nch2, CH), jnp.int32),
            pltpu.VMEM((nch2, CH), jnp.int32),
            pltpu.VMEM((CH, 8), jnp.float32),
            pltpu.VMEM_SHARED((N, 8), jnp.float32),
            pltpu.VMEM_SHARED((M, 8), jnp.float32),
            pltpu.VMEM_SHARED((NP_, 8), jnp.float32),
            pltpu.VMEM_SHARED((MP, 8), jnp.float32),
            pltpu.SemaphoreType.DMA,
        ],
        compiler_params=pltpu.CompilerParams(use_tc_tiling_on_sc=False),
        name="sc_degs",
    )


def _spmm(x, gidx2d, sidx2d, nrows):
    nnz = gidx2d.shape[0] * gidx2d.shape[1]
    zeros = jnp.zeros((nrows, x.shape[1]), jnp.float32)
    return _make_spmm(nnz, nrows, x.shape[1], x.shape[0])(
        x, gidx2d, sidx2d, zeros)


def _spmm_f(p, rdeg, gidx2d, sidx2d, nrows):
    nnz = gidx2d.shape[0] * gidx2d.shape[1]
    _, src_rows, d = p.shape
    zeros = jnp.zeros((nrows, d), jnp.float32)
    return _make_spmm(nnz, nrows, d, src_rows, True)(
        p, rdeg, gidx2d, sidx2d, zeros)


def _combine_mul_body(p_ref, rdeg_ref, o_ref):
    o_ref[...] = (p_ref[0] + p_ref[1]) * rdeg_ref[...]


def _combine_mul(p, rdeg):
    _, r, d = p.shape
    br = min(r, 2048)
    return pl.pallas_call(
        _combine_mul_body,
        grid=(r // br,),
        in_specs=[
            pl.BlockSpec((2, br, d), lambda i: (0, i, 0)),
            pl.BlockSpec((br, d), lambda i: (i, 0)),
        ],
        out_specs=pl.BlockSpec((br, d), lambda i: (i, 0)),
        out_shape=jax.ShapeDtypeStruct((r, d), jnp.float32),
    )(p, rdeg)


def _dense_tanh_body(p_ref, deg_ref, w_ref, b_ref, o_ref):
    dout = o_ref.shape[-1]
    pool = p_ref[0] + p_ref[1]
    z = jnp.dot(pool, w_ref[...], preferred_element_type=jnp.float32)
    o_ref[...] = jnp.tanh((z + b_ref[...]) / deg_ref[:, :dout])


def _dense_tanh(p, deg128, wmat, bvec):
    _, r, din = p.shape
    dout = wmat.shape[1]
    br = min(r, 2048)
    return pl.pallas_call(
        _dense_tanh_body,
        grid=(r // br,),
        in_specs=[
            pl.BlockSpec((2, br, din), lambda i: (0, i, 0)),
            pl.BlockSpec((br, 128), lambda i: (i, 0)),
            pl.BlockSpec((din, dout), lambda i: (0, 0)),
            pl.BlockSpec((1, dout), lambda i: (0, 0)),
        ],
        out_specs=pl.BlockSpec((br, dout), lambda i: (i, 0)),
        out_shape=jax.ShapeDtypeStruct((r, dout), jnp.float32),
    )(p, deg128, wmat, bvec.reshape(1, -1))


def _matmul_body(x_ref, w_ref, o_ref):
    o_ref[...] = jnp.dot(x_ref[...], w_ref[...],
                         preferred_element_type=jnp.float32)


def _matmul(x, w):
    r, din = x.shape
    dout = w.shape[1]
    br = min(r, 4096)
    return pl.pallas_call(
        _matmul_body,
        grid=(r // br,),
        in_specs=[
            pl.BlockSpec((br, din), lambda i: (i, 0)),
            pl.BlockSpec((din, dout), lambda i: (0, 0)),
        ],
        out_specs=pl.BlockSpec((br, dout), lambda i: (i, 0)),
        out_shape=jax.ShapeDtypeStruct((r, dout), jnp.float32),
    )(x, w)


def _bexp(width):
    lio = lax.broadcasted_iota(jnp.int32, (128, 16 * width), 0)
    mio = lax.broadcasted_iota(jnp.int32, (128, 16 * width), 1)
    return (lio == 8 * (mio // width)).astype(jnp.float32)


def _deg_expand_body(pn_ref, pm_ref, pnp_ref, pmp_ref,
                     on_ref, onp_ref, rm32_ref, rmp32_ref, rm8_ref,
                     rmp8_ref):
    b128 = _bexp(128)
    for p_ref, o_ref in ((pn_ref, on_ref), (pnp_ref, onp_ref)):
        x = p_ref[0] + p_ref[1]
        o_ref[...] = jnp.dot(x, b128,
                             preferred_element_type=jnp.float32) + 1.0
    b32 = _bexp(32)
    b8 = _bexp(8)
    for p_ref, o32_ref, o8_ref in ((pm_ref, rm32_ref, rm8_ref),
                                   (pmp_ref, rmp32_ref, rmp8_ref)):
        x = p_ref[0] + p_ref[1]
        o32_ref[...] = 1.0 / (jnp.dot(x, b32,
                                      preferred_element_type=jnp.float32)
                              + 1.0)
        o8_ref[...] = 1.0 / (jnp.dot(x, b8,
                                     preferred_element_type=jnp.float32)
                             + 1.0)


def _finalize_degs(pn, pm, pnp, pmp):
    outs = pl.pallas_call(
        _deg_expand_body,
        out_shape=[
            jax.ShapeDtypeStruct((N // 16, 2048), jnp.float32),
            jax.ShapeDtypeStruct((NP_ // 16, 2048), jnp.float32),
            jax.ShapeDtypeStruct((M // 16, 512), jnp.float32),
            jax.ShapeDtypeStruct((MP // 16, 512), jnp.float32),
            jax.ShapeDtypeStruct((M // 16, 128), jnp.float32),
            jax.ShapeDtypeStruct((MP // 16, 128), jnp.float32),
        ],
    )(*[p.reshape(2, p.shape[1] // 16, 128) for p in (pn, pm, pnp, pmp)])
    shp = ((N, 128), (NP_, 128), (M, 32), (MP, 32), (M, 8), (MP, 8))
    return [o.reshape(s) for o, s in zip(outs, shp)]


def _tail_body(c0_ref, c1_ref, c2_ref, c3c_ref, c3r_ref, k1t_ref, bk1_ref,
               k2t_ref, bk2_ref, wout_ref, bout_ref, o_ref):
    vrow = c3r_ref[...].reshape(1, N_PER)
    vcol = c3c_ref[...]
    jp = lax.broadcasted_iota(jnp.int32, (N_PER, N_PER), 1)
    jj = lax.broadcasted_iota(jnp.int32, (N_PER, N_PER), 0)
    ahead = (vrow > vcol) | ((vrow == vcol) & (jp < jj))
    rank = jnp.sum(ahead.astype(jnp.float32), axis=1, keepdims=True)
    kio = lax.broadcasted_iota(jnp.int32, (N_PER, K_SORT), 1).astype(
        jnp.float32)
    sel = (rank == kio).astype(jnp.float32)

    def pool_t(x):
        return lax.dot_general(sel, x, (((0,), (0,)), ((), ())),
                               preferred_element_type=jnp.float32)

    z1 = (jnp.dot(pool_t(c0_ref[...]), k1t_ref[0:32, :],
                  preferred_element_type=jnp.float32)
          + jnp.dot(pool_t(c1_ref[...]), k1t_ref[32:64, :],
                    preferred_element_type=jnp.float32)
          + jnp.dot(pool_t(c2_ref[...]), k1t_ref[64:96, :],
                    preferred_element_type=jnp.float32)
          + jnp.dot(pool_t(vcol), k1t_ref[96:97, :],
                    preferred_element_type=jnp.float32))
    z1 = jnp.maximum(z1 + bk1_ref[...], 0.0)
    wio = lax.broadcasted_iota(jnp.int32, (K_SORT, K_SORT // 2), 1)
    jio = lax.broadcasted_iota(jnp.int32, (K_SORT, K_SORT // 2), 0)
    s_even = (jio == 2 * wio).astype(jnp.float32)
    s_odd = (jio == 2 * wio + 1).astype(jnp.float32)

    def sel_t(smat):
        return lax.dot_general(smat, z1, (((0,), (0,)), ((), ())),
                               preferred_element_type=jnp.float32)

    zp = jnp.maximum(sel_t(s_even), sel_t(s_odd))
    c2 = jnp.zeros((11, 32), jnp.float32)
    for t in range(5):
        c2 = c2 + jnp.dot(zp[t:t + 11, :], k2t_ref[t],
                          preferred_element_type=jnp.float32)
    c2 = jnp.maximum(c2 + bk2_ref[...], 0.0)
    acc = jnp.zeros((1, 64), jnp.float32)
    for w in range(11):
        acc = acc + jnp.dot(c2[w:w + 1, :], wout_ref[w],
                            preferred_element_type=jnp.float32)
    o_ref[...] = jnp.maximum(acc + bout_ref[...], 0.0).reshape(1, 1, 64)


def _tail(c0, c1, c2, c3, k1t, bk1, k2t, bk2, woutr, bout):
    c3row = c3.reshape(G, 1, N_PER)
    grid = (G,)
    out = pl.pallas_call(
        _tail_body,
        grid=grid,
        in_specs=[
            pl.BlockSpec((N_PER, 32), lambda g: (g, 0)),
            pl.BlockSpec((N_PER, 32), lambda g: (g, 0)),
            pl.BlockSpec((N_PER, 32), lambda g: (g, 0)),
            pl.BlockSpec((N_PER, 1), lambda g: (g, 0)),
            pl.BlockSpec((1, 1, N_PER), lambda g: (g, 0, 0)),
            pl.BlockSpec((97, 16), lambda g: (0, 0)),
            pl.BlockSpec((1, 16), lambda g: (0, 0)),
            pl.BlockSpec((5, 16, 32), lambda g: (0, 0, 0)),
            pl.BlockSpec((1, 32), lambda g: (0, 0)),
            pl.BlockSpec((11, 32, 64), lambda g: (0, 0, 0)),
            pl.BlockSpec((1, 64), lambda g: (0, 0)),
        ],
        out_specs=pl.BlockSpec((1, 1, 64), lambda g: (g, 0, 0)),
        out_shape=jax.ShapeDtypeStruct((G, 1, 64), jnp.float32),
    )(c0, c1, c2, c3, c3row, k1t, bk1.reshape(1, 16), k2t,
      bk2.reshape(1, 32), woutr, bout.reshape(1, 64))
    return out.reshape(G, 64)


def kernel(node_feat, n2m_row, n2m_col, np2mp_row, np2mp_col, m2mp_row,
           m2mp_col, W0, b0, W1, b1, W2, b2, W3, b3, W4, b4, W5, b5, W6, b6,
           W7, b7, K1, bK1, K2, bK2, Wout, bout):
    r_n2m = n2m_row.reshape(-1, CH)
    c_n2m = n2m_col.reshape(-1, CH)
    r_np2mp = np2mp_row.reshape(-1, CH)
    c_np2mp = np2mp_col.reshape(-1, CH)
    r_m2mp = m2mp_row.reshape(-1, CH)
    c_m2mp = m2mp_col.reshape(-1, CH)

    ones8 = jnp.ones((CH, 8), jnp.float32)
    zeros8 = jnp.zeros((N // NS, 8), jnp.float32)
    pn, pm, pnp, pmp = _make_degs()(r_n2m, c_n2m, r_np2mp, c_np2mp,
                                    ones8, zeros8)
    (node_hdegs, node_hdegs_, rM32, rMP32, rM8,
     rMP8) = _finalize_degs(pn, pm, pnp, pmp)

    Ws = [(jnp.eye(32, dtype=jnp.float32), b0), (W1, b1), (W2, b2),
          (W3, b3), (W4, b4), (W5, b5),
          (jnp.pad(W6, ((0, 0), (0, 7))), jnp.pad(b6, (0, 7))),
          (jnp.pad(W7, ((0, 7), (0, 0))), b7)]

    cur = _matmul(node_feat, W0)
    cats = []
    lv = 0
    for it in range(4):
        p = _spmm(cur, r_n2m, c_n2m, M)
        p = _spmm_f(p, rM32, r_m2mp, c_m2mp, MP)
        p = _spmm_f(p, rMP32, c_np2mp, r_np2mp, NP_)
        wmat, bvec = Ws[lv]
        cur_ = _dense_tanh(p, node_hdegs_, wmat, bvec)
        lv += 1
        p = _spmm(cur_, r_np2mp, c_np2mp, MP)
        if it < 3:
            p = _spmm_f(p, rMP32, c_m2mp, r_m2mp, M)
            p = _spmm_f(p, rM32, c_n2m, r_n2m, N)
        else:
            a = _combine_mul(p, rMP8)
            p = _spmm(a, c_m2mp, r_m2mp, M)
            a = _combine_mul(p, rM8)
            p = _spmm(a, c_n2m, r_n2m, N)
        wmat, bvec = Ws[lv]
        cur = _dense_tanh(p, node_hdegs, wmat, bvec)
        lv += 1
        cats.append(cur)

    k1t = K1.T
    k2t = jnp.transpose(K2, (2, 1, 0))
    woutr = jnp.transpose(Wout.reshape(32, 11, 64), (1, 0, 2))
    return _tail(cats[0], cats[1], cats[2], cats[3], k1t, bK1, k2t, bK2,
                 woutr, bout)


# speedup vs baseline: 1.0290x; 1.0290x over previous
import functools

import jax
import jax.numpy as jnp
from jax import lax
from jax.experimental import pallas as pl
from jax.experimental.pallas import tpu as pltpu
from jax.experimental.pallas import tpu_sc as plsc

G = 64
N_PER = 512
N = G * N_PER
M = 8192
NP_ = 8192
MP = 4096
K_SORT = 30

NC = 2
NS = 16
NW = NC * NS
CH = 128
NNZ1 = 524288
NNZ2 = 131072


_SPMEM_BUDGET = 1966080


@functools.lru_cache(maxsize=None)
def _make_spmm(nnz, nrows, d, src_rows, fused=False):
    epw = nnz // NW
    nch = epw // CH
    rpw = nrows // NS
    spw = src_rows // NS
    comb = 3 * CH * d if fused else 0
    sh_extra = src_rows * d if fused else 0
    assert not fused or (d % 16 == 0 and spw % CH == 0)
    K = 0
    NB = nch
    for cand in (8, 4, 2):
        if nch % cand == 0 and (
                NS * (cand * CH * d + 2 * nch * CH + comb)
                + nrows * d + sh_extra <= _SPMEM_BUDGET):
            K = cand
            break
    if K < 8 and nch > 16:
        for cand in (8, 4, 3, 2):
            if (NS * (cand * CH * d + 4 * 16 * CH + comb)
                    + nrows * d + sh_extra <= _SPMEM_BUDGET):
                if cand > K:
                    K = cand
                    NB = 16
                break
    assert K >= 2 and nch % NB == 0, (nnz, nrows, d, fused)
    nblk = nch // NB
    NLANE = d // 16
    mesh = plsc.VectorSubcoreMesh(
        core_axis_name="c", subcore_axis_name="s", num_cores=NC,
        num_subcores=NS)

    GA = K // 2
    LG = K - GA

    def body(*refs):
        if fused:
            (x_hbm, rdeg_hbm, gidx_hbm, sidx_hbm, zeros_hbm, out_hbm,
             gidx_v, sidx_v, rows_v, cb_v, acc_sh, src_sh, gsem,
             ssem) = refs
        else:
            (x_hbm, gidx_hbm, sidx_hbm, zeros_hbm, out_hbm,
             gidx_v, sidx_v, rows_v, acc_sh, gsem, ssem) = refs
            src_sh = x_hbm
        c = lax.axis_index("c")
        s = lax.axis_index("s")
        w = c * NS + s
        r0 = s * rpw
        pltpu.sync_copy(zeros_hbm.at[pl.ds(r0, rpw), :],
                        acc_sh.at[pl.ds(r0, rpw), :])
        if fused:
            for t in range(spw // CH):
                rs = s * spw + t * CH
                pltpu.sync_copy(x_hbm.at[0, pl.ds(rs, CH), :], cb_v.at[0])
                pltpu.sync_copy(x_hbm.at[1, pl.ds(rs, CH), :], cb_v.at[1])
                pltpu.sync_copy(rdeg_hbm.at[pl.ds(rs, CH), :], cb_v.at[2])

                def vstep(i, carry):
                    for u in range(8):
                        ii = i * 8 + u
                        r = ii // NLANE
                        o = (ii % NLANE) * 16
                        cb_v[0, r, pl.ds(o, 16)] = (
                            (cb_v[0, r, pl.ds(o, 16)]
                             + cb_v[1, r, pl.ds(o, 16)])
                            * cb_v[2, r, pl.ds(o, 16)])
                    return carry

                lax.fori_loop(0, CH * NLANE // 8, vstep, 0)
                pltpu.sync_copy(cb_v.at[0], src_sh.at[pl.ds(rs, CH), :])
        base = w * nch

        def stage_idx(b, buf):
            pltpu.sync_copy(gidx_hbm.at[pl.ds(base + b * NB, NB), :],
                            gidx_v.at[buf])
            pltpu.sync_copy(sidx_hbm.at[pl.ds(base + b * NB, NB), :],
                            sidx_v.at[buf])

        stage_idx(0, 0)
        plsc.subcore_barrier()

        def gather(j):
            pltpu.async_copy(
                src_sh.at[gidx_v.at[(j // NB) % 2, j % NB]],
                rows_v.at[pl.ds((j % K) * CH, CH), :], gsem)

        def gather_wait(j):
            pltpu.make_async_copy(
                src_sh.at[gidx_v.at[0, 0]],
                rows_v.at[pl.ds((j % K) * CH, CH), :], gsem).wait()

        def scat(j):
            pltpu.async_copy(
                rows_v.at[pl.ds((j % K) * CH, CH), :],
                acc_sh.at[sidx_v.at[(j // NB) % 2, j % NB]], ssem, add=True)

        def scat_wait(j):
            pltpu.make_async_copy(
                rows_v.at[pl.ds((j % K) * CH, CH), :],
                acc_sh.at[sidx_v.at[0, 0]], ssem).wait()

        for j in range(GA):
            gather(j)

        def blk(b, carry):
            @pl.when(b + 1 < nblk)
            def _():
                stage_idx(b + 1, (b + 1) % 2)

            def step(q, carry2):
                j = b * NB + q

                @pl.when(q >= LG)
                def _():
                    scat_wait(j - LG)

                @pl.when(j + GA < nch)
                def _():
                    gather(j + GA)
                gather_wait(j)
                scat(j)
                return carry2

            lax.fori_loop(0, NB, step, carry)
            def drain(q, carry2):
                scat_wait(b * NB + NB - LG + q)
                return carry2

            lax.fori_loop(0, LG, drain, carry)
            return carry

        lax.fori_loop(0, nblk, blk, 0)
        plsc.subcore_barrier()
        pltpu.sync_copy(acc_sh.at[pl.ds(r0, rpw), :],
                        out_hbm.at[c, pl.ds(r0, rpw), :])

    scratch = [
        pltpu.VMEM((2, NB, CH), jnp.int32) if nblk > 1
        else pltpu.VMEM((1, NB, CH), jnp.int32),
        pltpu.VMEM((2, NB, CH), jnp.int32) if nblk > 1
        else pltpu.VMEM((1, NB, CH), jnp.int32),
        pltpu.VMEM((K * CH, d), jnp.float32),
    ]
    if fused:
        scratch.append(pltpu.VMEM((3, CH, d), jnp.float32))
    scratch.append(pltpu.VMEM_SHARED((nrows, d), jnp.float32))
    if fused:
        scratch.append(pltpu.VMEM_SHARED((src_rows, d), jnp.float32))
    scratch += [pltpu.SemaphoreType.DMA, pltpu.SemaphoreType.DMA]
    return pl.kernel(
        body,
        out_type=jax.ShapeDtypeStruct((NC, nrows, d), jnp.float32),
        mesh=mesh,
        scratch_types=scratch,
        compiler_params=pltpu.CompilerParams(use_tc_tiling_on_sc=False),
        name="sc_spmm%s_%d_%d_%d" % ("f" if fused else "", nnz, nrows, d),
    )


@functools.lru_cache(maxsize=None)
def _make_degs():
    nch1 = (NNZ1 // NW) // CH
    nch2 = (NNZ2 // NW) // CH
    sizes = (N, M, NP_, MP)
    mesh = plsc.VectorSubcoreMesh(
        core_axis_name="c", subcore_axis_name="s", num_cores=NC,
        num_subcores=NS)

    def body(rn_hbm, cn_hbm, rp_hbm, cp_hbm, ones_hbm, zeros_hbm,
             on_hbm, om_hbm, onp_hbm, omp_hbm,
             rn_v, cn_v, rp_v, cp_v, ones_v, an, am, anp, amp, sem):
        c = lax.axis_index("c")
        s = lax.axis_index("s")
        w = c * NS + s
        accs = (an, am, anp, amp)
        outs = (on_hbm, om_hbm, onp_hbm, omp_hbm)
        for acc, r in zip(accs, sizes):
            rp = r // NS
            pltpu.sync_copy(zeros_hbm.at[pl.ds(0, rp), :],
                            acc.at[pl.ds(s * rp, rp), :])
        pltpu.sync_copy(ones_hbm, ones_v)
        pltpu.sync_copy(rn_hbm.at[pl.ds(w * nch1, nch1), :], rn_v)
        pltpu.sync_copy(cn_hbm.at[pl.ds(w * nch1, nch1), :], cn_v)
        pltpu.sync_copy(rp_hbm.at[pl.ds(w * nch2, nch2), :], rp_v)
        pltpu.sync_copy(cp_hbm.at[pl.ds(w * nch2, nch2), :], cp_v)
        plsc.subcore_barrier()

        for idx_v, nchl, acc in ((rn_v, nch1, an), (cn_v, nch1, am),
                                 (rp_v, nch2, anp), (cp_v, nch2, amp)):
            def st(j, carry, idx_v=idx_v, acc=acc):
                pltpu.async_copy(ones_v, acc.at[idx_v.at[j]], sem, add=True)

                @pl.when(j >= 8)
                def _():
                    pltpu.make_async_copy(
                        ones_v, acc.at[idx_v.at[0]], sem).wait()
                return carry

            lax.fori_loop(0, nchl, st, 0)
            for _ in range(min(8, nchl)):
                pltpu.make_async_copy(ones_v, acc.at[idx_v.at[0]],
                                      sem).wait()
        plsc.subcore_barrier()
        for acc, r, out in zip(accs, sizes, outs):
            rp = r // NS
            pltpu.sync_copy(acc.at[pl.ds(s * rp, rp), :],
                            out.at[c, pl.ds(s * rp, rp), :])

    return pl.kernel(
        body,
        out_type=[jax.ShapeDtypeStruct((NC, r, 8), jnp.float32)
                  for r in sizes],
        mesh=mesh,
        scratch_types=[
            pltpu.VMEM((nch1, CH), jnp.int32),
            pltpu.VMEM((nch1, CH), jnp.int32),
            pltpu.VMEM((nch2, CH), jnp.int32),
            pltpu.VMEM((nch2, CH), jnp.int32),
            pltpu.VMEM((CH, 8), jnp.float32),
            pltpu.VMEM_SHARED((N, 8), jnp.float32),
            pltpu.VMEM_SHARED((M, 8), jnp.float32),
            pltpu.VMEM_SHARED((NP_, 8), jnp.float32),
            pltpu.VMEM_SHARED((MP, 8), jnp.float32),
            pltpu.SemaphoreType.DMA,
        ],
        compiler_params=pltpu.CompilerParams(use_tc_tiling_on_sc=False),
        name="sc_degs",
    )


def _spmm(x, gidx2d, sidx2d, nrows):
    nnz = gidx2d.shape[0] * gidx2d.shape[1]
    zeros = jnp.zeros((nrows, x.shape[1]), jnp.float32)
    return _make_spmm(nnz, nrows, x.shape[1], x.shape[0])(
        x, gidx2d, sidx2d, zeros)


def _spmm_f(p, rdeg, gidx2d, sidx2d, nrows):
    nnz = gidx2d.shape[0] * gidx2d.shape[1]
    _, src_rows, d = p.shape
    zeros = jnp.zeros((nrows, d), jnp.float32)
    return _make_spmm(nnz, nrows, d, src_rows, True)(
        p, rdeg, gidx2d, sidx2d, zeros)


def _combine_mul_body(p_ref, rdeg_ref, o_ref):
    o_ref[...] = (p_ref[0] + p_ref[1]) * rdeg_ref[...]


def _combine_mul(p, rdeg):
    _, r, d = p.shape
    br = min(r, 2048)
    return pl.pallas_call(
        _combine_mul_body,
        grid=(r // br,),
        in_specs=[
            pl.BlockSpec((2, br, d), lambda i: (0, i, 0)),
            pl.BlockSpec((br, d), lambda i: (i, 0)),
        ],
        out_specs=pl.BlockSpec((br, d), lambda i: (i, 0)),
        out_shape=jax.ShapeDtypeStruct((r, d), jnp.float32),
    )(p, rdeg)


def _dense_tanh_body(p_ref, deg_ref, w_ref, b_ref, o_ref):
    dout = o_ref.shape[-1]
    pool = p_ref[0] + p_ref[1]
    z = jnp.dot(pool, w_ref[...], preferred_element_type=jnp.float32)
    o_ref[...] = jnp.tanh((z + b_ref[...]) / deg_ref[:, :dout])


def _dense_tanh(p, deg128, wmat, bvec):
    _, r, din = p.shape
    dout = wmat.shape[1]
    br = min(r, 2048)
    return pl.pallas_call(
        _dense_tanh_body,
        grid=(r // br,),
        in_specs=[
            pl.BlockSpec((2, br, din), lambda i: (0, i, 0)),
            pl.BlockSpec((br, 128), lambda i: (i, 0)),
            pl.BlockSpec((din, dout), lambda i: (0, 0)),
            pl.BlockSpec((1, dout), lambda i: (0, 0)),
        ],
        out_specs=pl.BlockSpec((br, dout), lambda i: (i, 0)),
        out_shape=jax.ShapeDtypeStruct((r, dout), jnp.float32),
    )(p, deg128, wmat, bvec.reshape(1, -1))


def _matmul_body(x_ref, w_ref, o_ref):
    o_ref[...] = jnp.dot(x_ref[...], w_ref[...],
                         preferred_element_type=jnp.float32)


def _matmul(x, w):
    r, din = x.shape
    dout = w.shape[1]
    br = min(r, 4096)
    return pl.pallas_call(
        _matmul_body,
        grid=(r // br,),
        in_specs=[
            pl.BlockSpec((br, din), lambda i: (i, 0)),
            pl.BlockSpec((din, dout), lambda i: (0, 0)),
        ],
        out_specs=pl.BlockSpec((br, dout), lambda i: (i, 0)),
        out_shape=jax.ShapeDtypeStruct((r, dout), jnp.float32),
    )(x, w)


def _bexp(width):
    lio = lax.broadcasted_iota(jnp.int32, (128, 16 * width), 0)
    mio = lax.broadcasted_iota(jnp.int32, (128, 16 * width), 1)
    return (lio == 8 * (mio // width)).astype(jnp.float32)


def _deg_expand_body(pn_ref, pm_ref, pnp_ref, pmp_ref,
                     on_ref, onp_ref, rm32_ref, rmp32_ref, rm8_ref,
                     rmp8_ref):
    b128 = _bexp(128)
    for p_ref, o_ref in ((pn_ref, on_ref), (pnp_ref, onp_ref)):
        x = p_ref[0] + p_ref[1]
        o_ref[...] = jnp.dot(x, b128,
                             preferred_element_type=jnp.float32) + 1.0
    b32 = _bexp(32)
    b8 = _bexp(8)
    for p_ref, o32_ref, o8_ref in ((pm_ref, rm32_ref, rm8_ref),
                                   (pmp_ref, rmp32_ref, rmp8_ref)):
        x = p_ref[0] + p_ref[1]
        o32_ref[...] = 1.0 / (jnp.dot(x, b32,
                                      preferred_element_type=jnp.float32)
                              + 1.0)
        o8_ref[...] = 1.0 / (jnp.dot(x, b8,
                                     preferred_element_type=jnp.float32)
                             + 1.0)


def _finalize_degs(pn, pm, pnp, pmp):
    outs = pl.pallas_call(
        _deg_expand_body,
        out_shape=[
            jax.ShapeDtypeStruct((N // 16, 2048), jnp.float32),
            jax.ShapeDtypeStruct((NP_ // 16, 2048), jnp.float32),
            jax.ShapeDtypeStruct((M // 16, 512), jnp.float32),
            jax.ShapeDtypeStruct((MP // 16, 512), jnp.float32),
            jax.ShapeDtypeStruct((M // 16, 128), jnp.float32),
            jax.ShapeDtypeStruct((MP // 16, 128), jnp.float32),
        ],
    )(*[p.reshape(2, p.shape[1] // 16, 128) for p in (pn, pm, pnp, pmp)])
    shp = ((N, 128), (NP_, 128), (M, 32), (MP, 32), (M, 8), (MP, 8))
    return [o.reshape(s) for o, s in zip(outs, shp)]


def _tail_body(c0_ref, c1_ref, c2_ref, c3c_ref, c3r_ref, k1t_ref, bk1_ref,
               k2t_ref, bk2_ref, wout_ref, bout_ref, o_ref):
    vrow = c3r_ref[...].reshape(1, N_PER)
    vcol = c3c_ref[...]
    jp = lax.broadcasted_iota(jnp.int32, (N_PER, N_PER), 1)
    jj = lax.broadcasted_iota(jnp.int32, (N_PER, N_PER), 0)
    ahead = (vrow > vcol) | ((vrow == vcol) & (jp < jj))
    rank = jnp.sum(ahead.astype(jnp.float32), axis=1, keepdims=True)
    kio = lax.broadcasted_iota(jnp.int32, (N_PER, K_SORT), 1).astype(
        jnp.float32)
    sel = (rank == kio).astype(jnp.float32)

    def pool_t(x):
        return lax.dot_general(sel, x, (((0,), (0,)), ((), ())),
                               preferred_element_type=jnp.float32)

    z1 = (jnp.dot(pool_t(c0_ref[...]), k1t_ref[0:32, :],
                  preferred_element_type=jnp.float32)
          + jnp.dot(pool_t(c1_ref[...]), k1t_ref[32:64, :],
                    preferred_element_type=jnp.float32)
          + jnp.dot(pool_t(c2_ref[...]), k1t_ref[64:96, :],
                    preferred_element_type=jnp.float32)
          + jnp.dot(pool_t(vcol), k1t_ref[96:97, :],
                    preferred_element_type=jnp.float32))
    z1 = jnp.maximum(z1 + bk1_ref[...], 0.0)
    wio = lax.broadcasted_iota(jnp.int32, (K_SORT, K_SORT // 2), 1)
    jio = lax.broadcasted_iota(jnp.int32, (K_SORT, K_SORT // 2), 0)
    s_even = (jio == 2 * wio).astype(jnp.float32)
    s_odd = (jio == 2 * wio + 1).astype(jnp.float32)

    def sel_t(smat):
        return lax.dot_general(smat, z1, (((0,), (0,)), ((), ())),
                               preferred_element_type=jnp.float32)

    zp = jnp.maximum(sel_t(s_even), sel_t(s_odd))
    c2 = jnp.zeros((11, 32), jnp.float32)
    for t in range(5):
        c2 = c2 + jnp.dot(zp[t:t + 11, :], k2t_ref[t],
                          preferred_element_type=jnp.float32)
    c2 = jnp.maximum(c2 + bk2_ref[...], 0.0)
    acc = jnp.zeros((1, 64), jnp.float32)
    for w in range(11):
        acc = acc + jnp.dot(c2[w:w + 1, :], wout_ref[w],
                            preferred_element_type=jnp.float32)
    o_ref[...] = jnp.maximum(acc + bout_ref[...], 0.0).reshape(1, 1, 64)


def _tail(c0, c1, c2, c3, k1t, bk1, k2t, bk2, woutr, bout):
    c3row = c3.reshape(G, 1, N_PER)
    grid = (G,)
    out = pl.pallas_call(
        _tail_body,
        grid=grid,
        in_specs=[
            pl.BlockSpec((N_PER, 32), lambda g: (g, 0)),
            pl.BlockSpec((N_PER, 32), lambda g: (g, 0)),
            pl.BlockSpec((N_PER, 32), lambda g: (g, 0)),
            pl.BlockSpec((N_PER, 1), lambda g: (g, 0)),
            pl.BlockSpec((1, 1, N_PER), lambda g: (g, 0, 0)),
            pl.BlockSpec((97, 16), lambda g: (0, 0)),
            pl.BlockSpec((1, 16), lambda g: (0, 0)),
            pl.BlockSpec((5, 16, 32), lambda g: (0, 0, 0)),
            pl.BlockSpec((1, 32), lambda g: (0, 0)),
            pl.BlockSpec((11, 32, 64), lambda g: (0, 0, 0)),
            pl.BlockSpec((1, 64), lambda g: (0, 0)),
        ],
        out_specs=pl.BlockSpec((1, 1, 64), lambda g: (g, 0, 0)),
        out_shape=jax.ShapeDtypeStruct((G, 1, 64), jnp.float32),
    )(c0, c1, c2, c3, c3row, k1t, bk1.reshape(1, 16), k2t,
      bk2.reshape(1, 32), woutr, bout.reshape(1, 64))
    return out.reshape(G, 64)


def kernel(node_feat, n2m_row, n2m_col, np2mp_row, np2mp_col, m2mp_row,
           m2mp_col, W0, b0, W1, b1, W2, b2, W3, b3, W4, b4, W5, b5, W6, b6,
           W7, b7, K1, bK1, K2, bK2, Wout, bout):
    r_n2m = n2m_row.reshape(-1, CH)
    c_n2m = n2m_col.reshape(-1, CH)
    r_np2mp = np2mp_row.reshape(-1, CH)
    c_np2mp = np2mp_col.reshape(-1, CH)
    r_m2mp = m2mp_row.reshape(-1, CH)
    c_m2mp = m2mp_col.reshape(-1, CH)

    ones8 = jnp.ones((CH, 8), jnp.float32)
    zeros8 = jnp.zeros((N // NS, 8), jnp.float32)
    pn, pm, pnp, pmp = _make_degs()(r_n2m, c_n2m, r_np2mp, c_np2mp,
                                    ones8, zeros8)
    (node_hdegs, node_hdegs_, rM32, rMP32, rM8,
     rMP8) = _finalize_degs(pn, pm, pnp, pmp)

    Ws = [(jnp.eye(32, dtype=jnp.float32), b0), (W1, b1), (W2, b2),
          (W3, b3), (W4, b4), (W5, b5),
          (jnp.pad(W6, ((0, 0), (0, 7))), jnp.pad(b6, (0, 7))),
          (jnp.pad(W7, ((0, 7), (0, 0))), b7)]

    cur = _matmul(node_feat, W0)
    cats = []
    lv = 0
    for it in range(4):
        p = _spmm(cur, r_n2m, c_n2m, M)
        p = _spmm_f(p, rM32, r_m2mp, c_m2mp, MP)
        p = _spmm_f(p, rMP32, c_np2mp, r_np2mp, NP_)
        wmat, bvec = Ws[lv]
        cur_ = _dense_tanh(p, node_hdegs_, wmat, bvec)
        lv += 1
        p = _spmm(cur_, r_np2mp, c_np2mp, MP)
        if it < 3:
            p = _spmm_f(p, rMP32, c_m2mp, r_m2mp, M)
            p = _spmm_f(p, rM32, c_n2m, r_n2m, N)
        else:
            a = _combine_mul(p, rMP8)
            p = _spmm(a, c_m2mp, r_m2mp, M)
            a = _combine_mul(p, rM8)
            p = _spmm(a, c_n2m, r_n2m, N)
        wmat, bvec = Ws[lv]
        cur = _dense_tanh(p, node_hdegs, wmat, bvec)
        lv += 1
        cats.append(cur)

    k1t = K1.T
    k2t = jnp.transpose(K2, (2, 1, 0))
    woutr = jnp.transpose(Wout.reshape(32, 11, 64), (1, 0, 2))
    return _tail(cats[0], cats[1], cats[2], cats[3], k1t, bK1, k2t, bK2,
                 woutr, bout)


# speedup vs baseline: 1.0395x; 1.0102x over previous
import functools

import jax
import jax.numpy as jnp
from jax import lax
from jax.experimental import pallas as pl
from jax.experimental.pallas import tpu as pltpu
from jax.experimental.pallas import tpu_sc as plsc

G = 64
N_PER = 512
N = G * N_PER
M = 8192
NP_ = 8192
MP = 4096
K_SORT = 30

NC = 2
NS = 16
NW = NC * NS
CH = 128
NNZ1 = 524288
NNZ2 = 131072


_SPMEM_BUDGET = 1966080


@functools.lru_cache(maxsize=None)
def _make_spmm(nnz, nrows, d, src_rows, fused=False):
    epw = nnz // NW
    nch = epw // CH
    rpw = nrows // NS
    spw = src_rows // NS
    comb = 3 * CH * d if fused else 0
    sh_extra = src_rows * d if fused else 0
    assert not fused or (d % 16 == 0 and spw % CH == 0)
    K = 0
    NB = nch
    for cand in (8, 4, 2):
        if nch % cand == 0 and (
                NS * (cand * CH * d + 2 * nch * CH + comb)
                + nrows * d + sh_extra <= _SPMEM_BUDGET):
            K = cand
            break
    if K < 8 and nch > 16:
        for cand in (8, 4, 3, 2):
            if (NS * (cand * CH * d + 4 * 16 * CH + comb)
                    + nrows * d + sh_extra <= _SPMEM_BUDGET):
                if cand > K:
                    K = cand
                    NB = 16
                break
    assert K >= 2 and nch % NB == 0, (nnz, nrows, d, fused)
    nblk = nch // NB
    NLANE = d // 16
    mesh = plsc.VectorSubcoreMesh(
        core_axis_name="c", subcore_axis_name="s", num_cores=NC,
        num_subcores=NS)

    GA = K // 2
    LG = K - GA

    def body(*refs):
        if fused:
            (x_hbm, rdeg_hbm, gidx_hbm, sidx_hbm, zeros_hbm, out_hbm,
             gidx_v, sidx_v, rows_v, cb_v, acc_sh, src_sh, gsem,
             ssem) = refs
        else:
            (x_hbm, gidx_hbm, sidx_hbm, zeros_hbm, out_hbm,
             gidx_v, sidx_v, rows_v, acc_sh, gsem, ssem) = refs
            src_sh = x_hbm
        c = lax.axis_index("c")
        s = lax.axis_index("s")
        w = c * NS + s
        r0 = s * rpw
        pltpu.sync_copy(zeros_hbm.at[pl.ds(r0, rpw), :],
                        acc_sh.at[pl.ds(r0, rpw), :])
        if fused:
            for t in range(spw // CH):
                rs = s * spw + t * CH
                pltpu.sync_copy(x_hbm.at[0, pl.ds(rs, CH), :], cb_v.at[0])
                pltpu.sync_copy(x_hbm.at[1, pl.ds(rs, CH), :], cb_v.at[1])
                pltpu.sync_copy(rdeg_hbm.at[pl.ds(rs, CH), :], cb_v.at[2])

                def vstep(i, carry):
                    for u in range(8):
                        ii = i * 8 + u
                        r = ii // NLANE
                        o = (ii % NLANE) * 16
                        cb_v[0, r, pl.ds(o, 16)] = (
                            (cb_v[0, r, pl.ds(o, 16)]
                             + cb_v[1, r, pl.ds(o, 16)])
                            * cb_v[2, r, pl.ds(o, 16)])
                    return carry

                lax.fori_loop(0, CH * NLANE // 8, vstep, 0)
                pltpu.sync_copy(cb_v.at[0], src_sh.at[pl.ds(rs, CH), :])
        base = w * nch

        def stage_idx(b, buf):
            pltpu.sync_copy(gidx_hbm.at[pl.ds(base + b * NB, NB), :],
                            gidx_v.at[buf])
            pltpu.sync_copy(sidx_hbm.at[pl.ds(base + b * NB, NB), :],
                            sidx_v.at[buf])

        stage_idx(0, 0)
        plsc.subcore_barrier()

        def gather(j):
            pltpu.async_copy(
                src_sh.at[gidx_v.at[(j // NB) % 2, j % NB]],
                rows_v.at[pl.ds((j % K) * CH, CH), :], gsem)

        def gather_wait(j):
            pltpu.make_async_copy(
                src_sh.at[gidx_v.at[0, 0]],
                rows_v.at[pl.ds((j % K) * CH, CH), :], gsem).wait()

        def scat(j):
            pltpu.async_copy(
                rows_v.at[pl.ds((j % K) * CH, CH), :],
                acc_sh.at[sidx_v.at[(j // NB) % 2, j % NB]], ssem, add=True)

        def scat_wait(j):
            pltpu.make_async_copy(
                rows_v.at[pl.ds((j % K) * CH, CH), :],
                acc_sh.at[sidx_v.at[0, 0]], ssem).wait()

        for j in range(GA):
            gather(j)

        def blk(b, carry):
            @pl.when(b + 1 < nblk)
            def _():
                stage_idx(b + 1, (b + 1) % 2)

            def step(q, carry2):
                j = b * NB + q

                @pl.when(q >= LG)
                def _():
                    scat_wait(j - LG)

                @pl.when(j + GA < nch)
                def _():
                    gather(j + GA)
                gather_wait(j)
                scat(j)
                return carry2

            lax.fori_loop(0, NB, step, carry)
            def drain(q, carry2):
                scat_wait(b * NB + NB - LG + q)
                return carry2

            lax.fori_loop(0, LG, drain, carry)
            return carry

        lax.fori_loop(0, nblk, blk, 0)
        plsc.subcore_barrier()
        pltpu.sync_copy(acc_sh.at[pl.ds(r0, rpw), :],
                        out_hbm.at[c, pl.ds(r0, rpw), :])

    scratch = [
        pltpu.VMEM((2, NB, CH), jnp.int32) if nblk > 1
        else pltpu.VMEM((1, NB, CH), jnp.int32),
        pltpu.VMEM((2, NB, CH), jnp.int32) if nblk > 1
        else pltpu.VMEM((1, NB, CH), jnp.int32),
        pltpu.VMEM((K * CH, d), jnp.float32),
    ]
    if fused:
        scratch.append(pltpu.VMEM((3, CH, d), jnp.float32))
    scratch.append(pltpu.VMEM_SHARED((nrows, d), jnp.float32))
    if fused:
        scratch.append(pltpu.VMEM_SHARED((src_rows, d), jnp.float32))
    scratch += [pltpu.SemaphoreType.DMA, pltpu.SemaphoreType.DMA]
    return pl.kernel(
        body,
        out_type=jax.ShapeDtypeStruct((NC, nrows, d), jnp.float32),
        mesh=mesh,
        scratch_types=scratch,
        compiler_params=pltpu.CompilerParams(use_tc_tiling_on_sc=False),
        name="sc_spmm%s_%d_%d_%d" % ("f" if fused else "", nnz, nrows, d),
    )


@functools.lru_cache(maxsize=None)
def _make_degs():
    nch1 = (NNZ1 // NW) // CH
    nch2 = (NNZ2 // NW) // CH
    sizes = (N, M, NP_, MP)
    mesh = plsc.VectorSubcoreMesh(
        core_axis_name="c", subcore_axis_name="s", num_cores=NC,
        num_subcores=NS)

    def body(rn_hbm, cn_hbm, rp_hbm, cp_hbm, ones_hbm, zeros_hbm,
             on_hbm, om_hbm, onp_hbm, omp_hbm,
             rn_v, cn_v, rp_v, cp_v, ones_v, an, am, anp, amp, sem):
        c = lax.axis_index("c")
        s = lax.axis_index("s")
        w = c * NS + s
        accs = (an, am, anp, amp)
        outs = (on_hbm, om_hbm, onp_hbm, omp_hbm)
        for acc, r in zip(accs, sizes):
            rp = r // NS
            pltpu.sync_copy(zeros_hbm.at[pl.ds(0, rp), :],
                            acc.at[pl.ds(s * rp, rp), :])
        pltpu.sync_copy(ones_hbm, ones_v)
        pltpu.sync_copy(rn_hbm.at[pl.ds(w * nch1, nch1), :], rn_v)
        pltpu.sync_copy(cn_hbm.at[pl.ds(w * nch1, nch1), :], cn_v)
        pltpu.sync_copy(rp_hbm.at[pl.ds(w * nch2, nch2), :], rp_v)
        pltpu.sync_copy(cp_hbm.at[pl.ds(w * nch2, nch2), :], cp_v)
        plsc.subcore_barrier()

        for idx_v, nchl, acc in ((rn_v, nch1, an), (cn_v, nch1, am),
                                 (rp_v, nch2, anp), (cp_v, nch2, amp)):
            def st(j, carry, idx_v=idx_v, acc=acc):
                pltpu.async_copy(ones_v, acc.at[idx_v.at[j]], sem, add=True)

                @pl.when(j >= 8)
                def _():
                    pltpu.make_async_copy(
                        ones_v, acc.at[idx_v.at[0]], sem).wait()
                return carry

            lax.fori_loop(0, nchl, st, 0)
            for _ in range(min(8, nchl)):
                pltpu.make_async_copy(ones_v, acc.at[idx_v.at[0]],
                                      sem).wait()
        plsc.subcore_barrier()
        for acc, r, out in zip(accs, sizes, outs):
            rp = r // NS
            pltpu.sync_copy(acc.at[pl.ds(s * rp, rp), :],
                            out.at[c, pl.ds(s * rp, rp), :])

    return pl.kernel(
        body,
        out_type=[jax.ShapeDtypeStruct((NC, r, 8), jnp.float32)
                  for r in sizes],
        mesh=mesh,
        scratch_types=[
            pltpu.VMEM((nch1, CH), jnp.int32),
            pltpu.VMEM((nch1, CH), jnp.int32),
            pltpu.VMEM((nch2, CH), jnp.int32),
            pltpu.VMEM((nch2, CH), jnp.int32),
            pltpu.VMEM((CH, 8), jnp.float32),
            pltpu.VMEM_SHARED((N, 8), jnp.float32),
            pltpu.VMEM_SHARED((M, 8), jnp.float32),
            pltpu.VMEM_SHARED((NP_, 8), jnp.float32),
            pltpu.VMEM_SHARED((MP, 8), jnp.float32),
            pltpu.SemaphoreType.DMA,
        ],
        compiler_params=pltpu.CompilerParams(use_tc_tiling_on_sc=False),
        name="sc_degs",
    )


def _spmm(x, gidx2d, sidx2d, nrows):
    nnz = gidx2d.shape[0] * gidx2d.shape[1]
    zeros = jnp.zeros((nrows, x.shape[1]), jnp.float32)
    return _make_spmm(nnz, nrows, x.shape[1], x.shape[0])(
        x, gidx2d, sidx2d, zeros)


def _spmm_f(p, rdeg, gidx2d, sidx2d, nrows):
    nnz = gidx2d.shape[0] * gidx2d.shape[1]
    _, src_rows, d = p.shape
    zeros = jnp.zeros((nrows, d), jnp.float32)
    return _make_spmm(nnz, nrows, d, src_rows, True)(
        p, rdeg, gidx2d, sidx2d, zeros)


def _combine_mul_body(p_ref, rdeg_ref, o_ref):
    o_ref[...] = (p_ref[0] + p_ref[1]) * rdeg_ref[...]


def _combine_mul(p, rdeg):
    _, r, d = p.shape
    br = min(r, 2048)
    return pl.pallas_call(
        _combine_mul_body,
        grid=(r // br,),
        in_specs=[
            pl.BlockSpec((2, br, d), lambda i: (0, i, 0)),
            pl.BlockSpec((br, d), lambda i: (i, 0)),
        ],
        out_specs=pl.BlockSpec((br, d), lambda i: (i, 0)),
        out_shape=jax.ShapeDtypeStruct((r, d), jnp.float32),
    )(p, rdeg)


def _dense_tanh_body(p_ref, deg_ref, w_ref, b_ref, o_ref):
    dout = o_ref.shape[-1]
    pool = p_ref[0] + p_ref[1]
    z = jnp.dot(pool, w_ref[...], preferred_element_type=jnp.float32)
    o_ref[...] = jnp.tanh((z + b_ref[...]) / deg_ref[:, :dout])


def _dense_tanh(p, deg128, wmat, bvec):
    _, r, din = p.shape
    dout = wmat.shape[1]
    br = min(r, 2048)
    return pl.pallas_call(
        _dense_tanh_body,
        grid=(r // br,),
        in_specs=[
            pl.BlockSpec((2, br, din), lambda i: (0, i, 0)),
            pl.BlockSpec((br, 32), lambda i: (i, 0)),
            pl.BlockSpec((din, dout), lambda i: (0, 0)),
            pl.BlockSpec((1, dout), lambda i: (0, 0)),
        ],
        out_specs=pl.BlockSpec((br, dout), lambda i: (i, 0)),
        out_shape=jax.ShapeDtypeStruct((r, dout), jnp.float32),
    )(p, deg128, wmat, bvec.reshape(1, -1))


def _matmul_body(x_ref, w_ref, o_ref):
    o_ref[...] = jnp.dot(x_ref[...], w_ref[...],
                         preferred_element_type=jnp.float32)


def _matmul(x, w):
    r, din = x.shape
    dout = w.shape[1]
    br = min(r, 4096)
    return pl.pallas_call(
        _matmul_body,
        grid=(r // br,),
        in_specs=[
            pl.BlockSpec((br, din), lambda i: (i, 0)),
            pl.BlockSpec((din, dout), lambda i: (0, 0)),
        ],
        out_specs=pl.BlockSpec((br, dout), lambda i: (i, 0)),
        out_shape=jax.ShapeDtypeStruct((r, dout), jnp.float32),
    )(x, w)


def _bexp(width):
    lio = lax.broadcasted_iota(jnp.int32, (128, 16 * width), 0)
    mio = lax.broadcasted_iota(jnp.int32, (128, 16 * width), 1)
    return (lio == 8 * (mio // width)).astype(jnp.float32)


def _deg_expand_body(pn_ref, pm_ref, pnp_ref, pmp_ref,
                     on_ref, onp_ref, rm32_ref, rmp32_ref):
    b32 = _bexp(32)
    for p_ref, o_ref in ((pn_ref, on_ref), (pnp_ref, onp_ref)):
        x = p_ref[0] + p_ref[1]
        o_ref[...] = jnp.dot(x, b32,
                             preferred_element_type=jnp.float32) + 1.0
    for p_ref, o32_ref in ((pm_ref, rm32_ref), (pmp_ref, rmp32_ref)):
        x = p_ref[0] + p_ref[1]
        o32_ref[...] = 1.0 / (jnp.dot(x, b32,
                                      preferred_element_type=jnp.float32)
                              + 1.0)


def _finalize_degs(pn, pm, pnp, pmp):
    outs = pl.pallas_call(
        _deg_expand_body,
        out_shape=[
            jax.ShapeDtypeStruct((N // 16, 512), jnp.float32),
            jax.ShapeDtypeStruct((NP_ // 16, 512), jnp.float32),
            jax.ShapeDtypeStruct((M // 16, 512), jnp.float32),
            jax.ShapeDtypeStruct((MP // 16, 512), jnp.float32),
        ],
    )(*[p.reshape(2, p.shape[1] // 16, 128) for p in (pn, pm, pnp, pmp)])
    shp = ((N, 32), (NP_, 32), (M, 32), (MP, 32))
    return [o.reshape(s) for o, s in zip(outs, shp)]


def _tail_body(c0_ref, c1_ref, c2_ref, c3c_ref, c3r_ref, k1t_ref, bk1_ref,
               k2t_ref, bk2_ref, wout_ref, bout_ref, o_ref):
    vrow = c3r_ref[...].reshape(1, N_PER)
    vcol = c3c_ref[...]
    jp = lax.broadcasted_iota(jnp.int32, (N_PER, N_PER), 1)
    jj = lax.broadcasted_iota(jnp.int32, (N_PER, N_PER), 0)
    ahead = (vrow > vcol) | ((vrow == vcol) & (jp < jj))
    rank = jnp.sum(ahead.astype(jnp.float32), axis=1, keepdims=True)
    kio = lax.broadcasted_iota(jnp.int32, (N_PER, K_SORT), 1).astype(
        jnp.float32)
    sel = (rank == kio).astype(jnp.float32)

    def pool_t(x):
        return lax.dot_general(sel, x, (((0,), (0,)), ((), ())),
                               preferred_element_type=jnp.float32)

    z1 = (jnp.dot(pool_t(c0_ref[...]), k1t_ref[0:32, :],
                  preferred_element_type=jnp.float32)
          + jnp.dot(pool_t(c1_ref[...]), k1t_ref[32:64, :],
                    preferred_element_type=jnp.float32)
          + jnp.dot(pool_t(c2_ref[...]), k1t_ref[64:96, :],
                    preferred_element_type=jnp.float32)
          + jnp.dot(pool_t(vcol), k1t_ref[96:97, :],
                    preferred_element_type=jnp.float32))
    z1 = jnp.maximum(z1 + bk1_ref[...], 0.0)
    wio = lax.broadcasted_iota(jnp.int32, (K_SORT, K_SORT // 2), 1)
    jio = lax.broadcasted_iota(jnp.int32, (K_SORT, K_SORT // 2), 0)
    s_even = (jio == 2 * wio).astype(jnp.float32)
    s_odd = (jio == 2 * wio + 1).astype(jnp.float32)

    def sel_t(smat):
        return lax.dot_general(smat, z1, (((0,), (0,)), ((), ())),
                               preferred_element_type=jnp.float32)

    zp = jnp.maximum(sel_t(s_even), sel_t(s_odd))
    c2 = jnp.zeros((11, 32), jnp.float32)
    for t in range(5):
        c2 = c2 + jnp.dot(zp[t:t + 11, :], k2t_ref[t],
                          preferred_element_type=jnp.float32)
    c2 = jnp.maximum(c2 + bk2_ref[...], 0.0)
    acc = jnp.zeros((1, 64), jnp.float32)
    for w in range(11):
        acc = acc + jnp.dot(c2[w:w + 1, :], wout_ref[w],
                            preferred_element_type=jnp.float32)
    o_ref[...] = jnp.maximum(acc + bout_ref[...], 0.0).reshape(1, 1, 64)


def _tail(c0, c1, c2, c3, k1t, bk1, k2t, bk2, woutr, bout):
    c3row = c3.reshape(G, 1, N_PER)
    grid = (G,)
    out = pl.pallas_call(
        _tail_body,
        grid=grid,
        in_specs=[
            pl.BlockSpec((N_PER, 32), lambda g: (g, 0)),
            pl.BlockSpec((N_PER, 32), lambda g: (g, 0)),
            pl.BlockSpec((N_PER, 32), lambda g: (g, 0)),
            pl.BlockSpec((N_PER, 1), lambda g: (g, 0)),
            pl.BlockSpec((1, 1, N_PER), lambda g: (g, 0, 0)),
            pl.BlockSpec((97, 16), lambda g: (0, 0)),
            pl.BlockSpec((1, 16), lambda g: (0, 0)),
            pl.BlockSpec((5, 16, 32), lambda g: (0, 0, 0)),
            pl.BlockSpec((1, 32), lambda g: (0, 0)),
            pl.BlockSpec((11, 32, 64), lambda g: (0, 0, 0)),
            pl.BlockSpec((1, 64), lambda g: (0, 0)),
        ],
        out_specs=pl.BlockSpec((1, 1, 64), lambda g: (g, 0, 0)),
        out_shape=jax.ShapeDtypeStruct((G, 1, 64), jnp.float32),
    )(c0, c1, c2, c3, c3row, k1t, bk1.reshape(1, 16), k2t,
      bk2.reshape(1, 32), woutr, bout.reshape(1, 64))
    return out.reshape(G, 64)


def kernel(node_feat, n2m_row, n2m_col, np2mp_row, np2mp_col, m2mp_row,
           m2mp_col, W0, b0, W1, b1, W2, b2, W3, b3, W4, b4, W5, b5, W6, b6,
           W7, b7, K1, bK1, K2, bK2, Wout, bout):
    r_n2m = n2m_row.reshape(-1, CH)
    c_n2m = n2m_col.reshape(-1, CH)
    r_np2mp = np2mp_row.reshape(-1, CH)
    c_np2mp = np2mp_col.reshape(-1, CH)
    r_m2mp = m2mp_row.reshape(-1, CH)
    c_m2mp = m2mp_col.reshape(-1, CH)

    ones8 = jnp.ones((CH, 8), jnp.float32)
    zeros8 = jnp.zeros((N // NS, 8), jnp.float32)
    pn, pm, pnp, pmp = _make_degs()(r_n2m, c_n2m, r_np2mp, c_np2mp,
                                    ones8, zeros8)
    node_hdegs, node_hdegs_, rM32, rMP32 = _finalize_degs(pn, pm, pnp, pmp)

    Ws = [(jnp.eye(32, dtype=jnp.float32), b0), (W1, b1), (W2, b2),
          (W3, b3), (W4, b4), (W5, b5),
          (jnp.pad(W6, ((0, 0), (0, 31))), jnp.pad(b6, (0, 31))),
          (jnp.pad(W7, ((0, 31), (0, 0))), b7)]

    cur = _matmul(node_feat, W0)
    cats = []
    lv = 0
    for it in range(4):
        p = _spmm(cur, r_n2m, c_n2m, M)
        p = _spmm_f(p, rM32, r_m2mp, c_m2mp, MP)
        p = _spmm_f(p, rMP32, c_np2mp, r_np2mp, NP_)
        wmat, bvec = Ws[lv]
        cur_ = _dense_tanh(p, node_hdegs_, wmat, bvec)
        lv += 1
        p = _spmm(cur_, r_np2mp, c_np2mp, MP)
        p = _spmm_f(p, rMP32, c_m2mp, r_m2mp, M)
        p = _spmm_f(p, rM32, c_n2m, r_n2m, N)
        wmat, bvec = Ws[lv]
        cur = _dense_tanh(p, node_hdegs, wmat, bvec)
        lv += 1
        cats.append(cur)

    k1t = K1.T
    k2t = jnp.transpose(K2, (2, 1, 0))
    woutr = jnp.transpose(Wout.reshape(32, 11, 64), (1, 0, 2))
    return _tail(cats[0], cats[1], cats[2], cats[3], k1t, bK1, k2t, bK2,
                 woutr, bout)


# speedup vs baseline: 1.0747x; 1.0339x over previous
import functools

import jax
import jax.numpy as jnp
from jax import lax
from jax.experimental import pallas as pl
from jax.experimental.pallas import tpu as pltpu
from jax.experimental.pallas import tpu_sc as plsc

G = 64
N_PER = 512
N = G * N_PER
M = 8192
NP_ = 8192
MP = 4096
K_SORT = 30

NC = 2
NS = 16
NW = NC * NS
CH = 128
NNZ1 = 524288
NNZ2 = 131072


_SPMEM_BUDGET = 1966080


@functools.lru_cache(maxsize=None)
def _make_spmm(nnz, nrows, d, src_rows, fused=False):
    epw = nnz // NW
    nch = epw // CH
    rpw = nrows // NS
    spw = src_rows // NS
    comb = 3 * CH * d if fused else 0
    sh_extra = 0
    assert not fused or (d % 16 == 0 and spw % CH == 0)
    K = 0
    NB = nch
    for cand in (8, 4, 2):
        if nch % cand == 0 and (
                NS * (cand * CH * d + 2 * nch * CH + comb)
                + nrows * d + sh_extra <= _SPMEM_BUDGET):
            K = cand
            break
    if K < 8 and nch > 16:
        for cand in (8, 4, 3, 2):
            if (NS * (cand * CH * d + 4 * 16 * CH + comb)
                    + nrows * d + sh_extra <= _SPMEM_BUDGET):
                if cand > K:
                    K = cand
                    NB = 16
                break
    assert K >= 2 and nch % NB == 0, (nnz, nrows, d, fused)
    nblk = nch // NB
    NLANE = d // 16
    mesh = plsc.VectorSubcoreMesh(
        core_axis_name="c", subcore_axis_name="s", num_cores=NC,
        num_subcores=NS)

    GA = K // 2
    LG = K - GA

    def body(*refs):
        if fused:
            (x_hbm, rdeg_hbm, gidx_hbm, sidx_hbm, zeros_hbm, out_hbm,
             src2_hbm, gidx_v, sidx_v, rows_v, cb_v, acc_sh, gsem,
             ssem) = refs
        else:
            (x_hbm, gidx_hbm, sidx_hbm, zeros_hbm, out_hbm,
             gidx_v, sidx_v, rows_v, acc_sh, gsem, ssem) = refs
        c = lax.axis_index("c")
        s = lax.axis_index("s")
        w = c * NS + s
        r0 = s * rpw
        pltpu.sync_copy(zeros_hbm.at[pl.ds(r0, rpw), :],
                        acc_sh.at[pl.ds(r0, rpw), :])
        if fused:
            src_sh = src2_hbm.at[c]
            for t in range(spw // CH):
                rs = s * spw + t * CH
                pltpu.sync_copy(x_hbm.at[0, pl.ds(rs, CH), :], cb_v.at[0])
                pltpu.sync_copy(x_hbm.at[1, pl.ds(rs, CH), :], cb_v.at[1])
                pltpu.sync_copy(rdeg_hbm.at[pl.ds(rs, CH), :], cb_v.at[2])

                def vstep(i, carry):
                    for u in range(8):
                        ii = i * 8 + u
                        r = ii // NLANE
                        o = (ii % NLANE) * 16
                        cb_v[0, r, pl.ds(o, 16)] = (
                            (cb_v[0, r, pl.ds(o, 16)]
                             + cb_v[1, r, pl.ds(o, 16)])
                            * cb_v[2, r, pl.ds(o, 16)])
                    return carry

                lax.fori_loop(0, CH * NLANE // 8, vstep, 0)
                pltpu.sync_copy(cb_v.at[0], src_sh.at[pl.ds(rs, CH), :])
        else:
            src_sh = x_hbm
        base = w * nch

        def stage_idx(b, buf):
            pltpu.sync_copy(gidx_hbm.at[pl.ds(base + b * NB, NB), :],
                            gidx_v.at[buf])
            pltpu.sync_copy(sidx_hbm.at[pl.ds(base + b * NB, NB), :],
                            sidx_v.at[buf])

        stage_idx(0, 0)
        plsc.subcore_barrier()

        def gather(j):
            pltpu.async_copy(
                src_sh.at[gidx_v.at[(j // NB) % 2, j % NB]],
                rows_v.at[pl.ds((j % K) * CH, CH), :], gsem)

        def gather_wait(j):
            pltpu.make_async_copy(
                src_sh.at[gidx_v.at[0, 0]],
                rows_v.at[pl.ds((j % K) * CH, CH), :], gsem).wait()

        def scat(j):
            pltpu.async_copy(
                rows_v.at[pl.ds((j % K) * CH, CH), :],
                acc_sh.at[sidx_v.at[(j // NB) % 2, j % NB]], ssem, add=True)

        def scat_wait(j):
            pltpu.make_async_copy(
                rows_v.at[pl.ds((j % K) * CH, CH), :],
                acc_sh.at[sidx_v.at[0, 0]], ssem).wait()

        for j in range(GA):
            gather(j)

        def blk(b, carry):
            @pl.when(b + 1 < nblk)
            def _():
                stage_idx(b + 1, (b + 1) % 2)

            def step(q, carry2):
                j = b * NB + q

                @pl.when(q >= LG)
                def _():
                    scat_wait(j - LG)

                @pl.when(j + GA < nch)
                def _():
                    gather(j + GA)
                gather_wait(j)
                scat(j)
                return carry2

            lax.fori_loop(0, NB, step, carry)
            def drain(q, carry2):
                scat_wait(b * NB + NB - LG + q)
                return carry2

            lax.fori_loop(0, LG, drain, carry)
            return carry

        lax.fori_loop(0, nblk, blk, 0)
        plsc.subcore_barrier()
        pltpu.sync_copy(acc_sh.at[pl.ds(r0, rpw), :],
                        out_hbm.at[c, pl.ds(r0, rpw), :])

    scratch = [
        pltpu.VMEM((2, NB, CH), jnp.int32) if nblk > 1
        else pltpu.VMEM((1, NB, CH), jnp.int32),
        pltpu.VMEM((2, NB, CH), jnp.int32) if nblk > 1
        else pltpu.VMEM((1, NB, CH), jnp.int32),
        pltpu.VMEM((K * CH, d), jnp.float32),
    ]
    if fused:
        scratch.append(pltpu.VMEM((3, CH, d), jnp.float32))
    scratch.append(pltpu.VMEM_SHARED((nrows, d), jnp.float32))
    scratch += [pltpu.SemaphoreType.DMA, pltpu.SemaphoreType.DMA]
    out_type = jax.ShapeDtypeStruct((NC, nrows, d), jnp.float32)
    if fused:
        out_type = [out_type,
                    jax.ShapeDtypeStruct((NC, src_rows, d), jnp.float32)]
    return pl.kernel(
        body,
        out_type=out_type,
        mesh=mesh,
        scratch_types=scratch,
        compiler_params=pltpu.CompilerParams(use_tc_tiling_on_sc=False),
        name="sc_spmm%s_%d_%d_%d" % ("f" if fused else "", nnz, nrows, d),
    )


@functools.lru_cache(maxsize=None)
def _make_degs():
    nch1 = (NNZ1 // NW) // CH
    nch2 = (NNZ2 // NW) // CH
    sizes = (N, M, NP_, MP)
    mesh = plsc.VectorSubcoreMesh(
        core_axis_name="c", subcore_axis_name="s", num_cores=NC,
        num_subcores=NS)

    def body(rn_hbm, cn_hbm, rp_hbm, cp_hbm, ones_hbm, zeros_hbm,
             on_hbm, om_hbm, onp_hbm, omp_hbm,
             rn_v, cn_v, rp_v, cp_v, ones_v, an, am, anp, amp, sem):
        c = lax.axis_index("c")
        s = lax.axis_index("s")
        w = c * NS + s
        accs = (an, am, anp, amp)
        outs = (on_hbm, om_hbm, onp_hbm, omp_hbm)
        for acc, r in zip(accs, sizes):
            rp = r // NS
            pltpu.sync_copy(zeros_hbm.at[pl.ds(0, rp), :],
                            acc.at[pl.ds(s * rp, rp), :])
        pltpu.sync_copy(ones_hbm, ones_v)
        pltpu.sync_copy(rn_hbm.at[pl.ds(w * nch1, nch1), :], rn_v)
        pltpu.sync_copy(cn_hbm.at[pl.ds(w * nch1, nch1), :], cn_v)
        pltpu.sync_copy(rp_hbm.at[pl.ds(w * nch2, nch2), :], rp_v)
        pltpu.sync_copy(cp_hbm.at[pl.ds(w * nch2, nch2), :], cp_v)
        plsc.subcore_barrier()

        for idx_v, nchl, acc in ((rn_v, nch1, an), (cn_v, nch1, am),
                                 (rp_v, nch2, anp), (cp_v, nch2, amp)):
            def st(j, carry, idx_v=idx_v, acc=acc):
                pltpu.async_copy(ones_v, acc.at[idx_v.at[j]], sem, add=True)

                @pl.when(j >= 8)
                def _():
                    pltpu.make_async_copy(
                        ones_v, acc.at[idx_v.at[0]], sem).wait()
                return carry

            lax.fori_loop(0, nchl, st, 0)
            for _ in range(min(8, nchl)):
                pltpu.make_async_copy(ones_v, acc.at[idx_v.at[0]],
                                      sem).wait()
        plsc.subcore_barrier()
        for acc, r, out in zip(accs, sizes, outs):
            rp = r // NS
            pltpu.sync_copy(acc.at[pl.ds(s * rp, rp), :],
                            out.at[c, pl.ds(s * rp, rp), :])

    return pl.kernel(
        body,
        out_type=[jax.ShapeDtypeStruct((NC, r, 8), jnp.float32)
                  for r in sizes],
        mesh=mesh,
        scratch_types=[
            pltpu.VMEM((nch1, CH), jnp.int32),
            pltpu.VMEM((nch1, CH), jnp.int32),
            pltpu.VMEM((nch2, CH), jnp.int32),
            pltpu.VMEM((nch2, CH), jnp.int32),
            pltpu.VMEM((CH, 8), jnp.float32),
            pltpu.VMEM_SHARED((N, 8), jnp.float32),
            pltpu.VMEM_SHARED((M, 8), jnp.float32),
            pltpu.VMEM_SHARED((NP_, 8), jnp.float32),
            pltpu.VMEM_SHARED((MP, 8), jnp.float32),
            pltpu.SemaphoreType.DMA,
        ],
        compiler_params=pltpu.CompilerParams(use_tc_tiling_on_sc=False),
        name="sc_degs",
    )


def _spmm(x, gidx2d, sidx2d, nrows):
    nnz = gidx2d.shape[0] * gidx2d.shape[1]
    zeros = jnp.zeros((nrows, x.shape[1]), jnp.float32)
    return _make_spmm(nnz, nrows, x.shape[1], x.shape[0])(
        x, gidx2d, sidx2d, zeros)


def _spmm_f(p, rdeg, gidx2d, sidx2d, nrows):
    nnz = gidx2d.shape[0] * gidx2d.shape[1]
    _, src_rows, d = p.shape
    zeros = jnp.zeros((nrows, d), jnp.float32)
    return _make_spmm(nnz, nrows, d, src_rows, True)(
        p, rdeg, gidx2d, sidx2d, zeros)[0]


def _combine_mul_body(p_ref, rdeg_ref, o_ref):
    o_ref[...] = (p_ref[0] + p_ref[1]) * rdeg_ref[...]


def _combine_mul(p, rdeg):
    _, r, d = p.shape
    br = min(r, 2048)
    return pl.pallas_call(
        _combine_mul_body,
        grid=(r // br,),
        in_specs=[
            pl.BlockSpec((2, br, d), lambda i: (0, i, 0)),
            pl.BlockSpec((br, d), lambda i: (i, 0)),
        ],
        out_specs=pl.BlockSpec((br, d), lambda i: (i, 0)),
        out_shape=jax.ShapeDtypeStruct((r, d), jnp.float32),
    )(p, rdeg)


def _dense_tanh_body(p_ref, deg_ref, w_ref, b_ref, o_ref):
    dout = o_ref.shape[-1]
    pool = p_ref[0] + p_ref[1]
    z = jnp.dot(pool, w_ref[...], preferred_element_type=jnp.float32)
    o_ref[...] = jnp.tanh((z + b_ref[...]) / deg_ref[:, :dout])


def _dense_tanh(p, deg128, wmat, bvec):
    _, r, din = p.shape
    dout = wmat.shape[1]
    br = min(r, 2048)
    return pl.pallas_call(
        _dense_tanh_body,
        grid=(r // br,),
        in_specs=[
            pl.BlockSpec((2, br, din), lambda i: (0, i, 0)),
            pl.BlockSpec((br, 32), lambda i: (i, 0)),
            pl.BlockSpec((din, dout), lambda i: (0, 0)),
            pl.BlockSpec((1, dout), lambda i: (0, 0)),
        ],
        out_specs=pl.BlockSpec((br, dout), lambda i: (i, 0)),
        out_shape=jax.ShapeDtypeStruct((r, dout), jnp.float32),
    )(p, deg128, wmat, bvec.reshape(1, -1))


def _matmul_body(x_ref, w_ref, o_ref):
    o_ref[...] = jnp.dot(x_ref[...], w_ref[...],
                         preferred_element_type=jnp.float32)


def _matmul(x, w):
    r, din = x.shape
    dout = w.shape[1]
    br = min(r, 4096)
    return pl.pallas_call(
        _matmul_body,
        grid=(r // br,),
        in_specs=[
            pl.BlockSpec((br, din), lambda i: (i, 0)),
            pl.BlockSpec((din, dout), lambda i: (0, 0)),
        ],
        out_specs=pl.BlockSpec((br, dout), lambda i: (i, 0)),
        out_shape=jax.ShapeDtypeStruct((r, dout), jnp.float32),
    )(x, w)


def _bexp(width):
    lio = lax.broadcasted_iota(jnp.int32, (128, 16 * width), 0)
    mio = lax.broadcasted_iota(jnp.int32, (128, 16 * width), 1)
    return (lio == 8 * (mio // width)).astype(jnp.float32)


def _deg_expand_body(pn_ref, pm_ref, pnp_ref, pmp_ref,
                     on_ref, onp_ref, rm32_ref, rmp32_ref):
    b32 = _bexp(32)
    for p_ref, o_ref in ((pn_ref, on_ref), (pnp_ref, onp_ref)):
        x = p_ref[0] + p_ref[1]
        o_ref[...] = jnp.dot(x, b32,
                             preferred_element_type=jnp.float32) + 1.0
    for p_ref, o32_ref in ((pm_ref, rm32_ref), (pmp_ref, rmp32_ref)):
        x = p_ref[0] + p_ref[1]
        o32_ref[...] = 1.0 / (jnp.dot(x, b32,
                                      preferred_element_type=jnp.float32)
                              + 1.0)


def _finalize_degs(pn, pm, pnp, pmp):
    outs = pl.pallas_call(
        _deg_expand_body,
        out_shape=[
            jax.ShapeDtypeStruct((N // 16, 512), jnp.float32),
            jax.ShapeDtypeStruct((NP_ // 16, 512), jnp.float32),
            jax.ShapeDtypeStruct((M // 16, 512), jnp.float32),
            jax.ShapeDtypeStruct((MP // 16, 512), jnp.float32),
        ],
    )(*[p.reshape(2, p.shape[1] // 16, 128) for p in (pn, pm, pnp, pmp)])
    shp = ((N, 32), (NP_, 32), (M, 32), (MP, 32))
    return [o.reshape(s) for o, s in zip(outs, shp)]


def _tail_body(c0_ref, c1_ref, c2_ref, c3c_ref, c3r_ref, k1t_ref, bk1_ref,
               k2t_ref, bk2_ref, wout_ref, bout_ref, o_ref):
    vrow = c3r_ref[...].reshape(1, N_PER)
    vcol = c3c_ref[...]
    jp = lax.broadcasted_iota(jnp.int32, (N_PER, N_PER), 1)
    jj = lax.broadcasted_iota(jnp.int32, (N_PER, N_PER), 0)
    ahead = (vrow > vcol) | ((vrow == vcol) & (jp < jj))
    rank = jnp.sum(ahead.astype(jnp.float32), axis=1, keepdims=True)
    kio = lax.broadcasted_iota(jnp.int32, (N_PER, K_SORT), 1).astype(
        jnp.float32)
    sel = (rank == kio).astype(jnp.float32)

    def pool_t(x):
        return lax.dot_general(sel, x, (((0,), (0,)), ((), ())),
                               preferred_element_type=jnp.float32)

    z1 = (jnp.dot(pool_t(c0_ref[...]), k1t_ref[0:32, :],
                  preferred_element_type=jnp.float32)
          + jnp.dot(pool_t(c1_ref[...]), k1t_ref[32:64, :],
                    preferred_element_type=jnp.float32)
          + jnp.dot(pool_t(c2_ref[...]), k1t_ref[64:96, :],
                    preferred_element_type=jnp.float32)
          + jnp.dot(pool_t(vcol), k1t_ref[96:97, :],
                    preferred_element_type=jnp.float32))
    z1 = jnp.maximum(z1 + bk1_ref[...], 0.0)
    wio = lax.broadcasted_iota(jnp.int32, (K_SORT, K_SORT // 2), 1)
    jio = lax.broadcasted_iota(jnp.int32, (K_SORT, K_SORT // 2), 0)
    s_even = (jio == 2 * wio).astype(jnp.float32)
    s_odd = (jio == 2 * wio + 1).astype(jnp.float32)

    def sel_t(smat):
        return lax.dot_general(smat, z1, (((0,), (0,)), ((), ())),
                               preferred_element_type=jnp.float32)

    zp = jnp.maximum(sel_t(s_even), sel_t(s_odd))
    c2 = jnp.zeros((11, 32), jnp.float32)
    for t in range(5):
        c2 = c2 + jnp.dot(zp[t:t + 11, :], k2t_ref[t],
                          preferred_element_type=jnp.float32)
    c2 = jnp.maximum(c2 + bk2_ref[...], 0.0)
    acc = jnp.zeros((1, 64), jnp.float32)
    for w in range(11):
        acc = acc + jnp.dot(c2[w:w + 1, :], wout_ref[w],
                            preferred_element_type=jnp.float32)
    o_ref[...] = jnp.maximum(acc + bout_ref[...], 0.0).reshape(1, 1, 64)


def _tail(c0, c1, c2, c3, k1t, bk1, k2t, bk2, woutr, bout):
    c3row = c3.reshape(G, 1, N_PER)
    grid = (G,)
    out = pl.pallas_call(
        _tail_body,
        grid=grid,
        in_specs=[
            pl.BlockSpec((N_PER, 32), lambda g: (g, 0)),
            pl.BlockSpec((N_PER, 32), lambda g: (g, 0)),
            pl.BlockSpec((N_PER, 32), lambda g: (g, 0)),
            pl.BlockSpec((N_PER, 1), lambda g: (g, 0)),
            pl.BlockSpec((1, 1, N_PER), lambda g: (g, 0, 0)),
            pl.BlockSpec((97, 16), lambda g: (0, 0)),
            pl.BlockSpec((1, 16), lambda g: (0, 0)),
            pl.BlockSpec((5, 16, 32), lambda g: (0, 0, 0)),
            pl.BlockSpec((1, 32), lambda g: (0, 0)),
            pl.BlockSpec((11, 32, 64), lambda g: (0, 0, 0)),
            pl.BlockSpec((1, 64), lambda g: (0, 0)),
        ],
        out_specs=pl.BlockSpec((1, 1, 64), lambda g: (g, 0, 0)),
        out_shape=jax.ShapeDtypeStruct((G, 1, 64), jnp.float32),
    )(c0, c1, c2, c3, c3row, k1t, bk1.reshape(1, 16), k2t,
      bk2.reshape(1, 32), woutr, bout.reshape(1, 64))
    return out.reshape(G, 64)


def kernel(node_feat, n2m_row, n2m_col, np2mp_row, np2mp_col, m2mp_row,
           m2mp_col, W0, b0, W1, b1, W2, b2, W3, b3, W4, b4, W5, b5, W6, b6,
           W7, b7, K1, bK1, K2, bK2, Wout, bout):
    r_n2m = n2m_row.reshape(-1, CH)
    c_n2m = n2m_col.reshape(-1, CH)
    r_np2mp = np2mp_row.reshape(-1, CH)
    c_np2mp = np2mp_col.reshape(-1, CH)
    r_m2mp = m2mp_row.reshape(-1, CH)
    c_m2mp = m2mp_col.reshape(-1, CH)

    ones8 = jnp.ones((CH, 8), jnp.float32)
    zeros8 = jnp.zeros((N // NS, 8), jnp.float32)
    pn, pm, pnp, pmp = _make_degs()(r_n2m, c_n2m, r_np2mp, c_np2mp,
                                    ones8, zeros8)
    node_hdegs, node_hdegs_, rM32, rMP32 = _finalize_degs(pn, pm, pnp, pmp)

    Ws = [(jnp.eye(32, dtype=jnp.float32), b0), (W1, b1), (W2, b2),
          (W3, b3), (W4, b4), (W5, b5),
          (jnp.pad(W6, ((0, 0), (0, 31))), jnp.pad(b6, (0, 31))),
          (jnp.pad(W7, ((0, 31), (0, 0))), b7)]

    cur = _matmul(node_feat, W0)
    cats = []
    lv = 0
    for it in range(4):
        p = _spmm(cur, r_n2m, c_n2m, M)
        p = _spmm_f(p, rM32, r_m2mp, c_m2mp, MP)
        p = _spmm_f(p, rMP32, c_np2mp, r_np2mp, NP_)
        wmat, bvec = Ws[lv]
        cur_ = _dense_tanh(p, node_hdegs_, wmat, bvec)
        lv += 1
        p = _spmm(cur_, r_np2mp, c_np2mp, MP)
        p = _spmm_f(p, rMP32, c_m2mp, r_m2mp, M)
        p = _spmm_f(p, rM32, c_n2m, r_n2m, N)
        wmat, bvec = Ws[lv]
        cur = _dense_tanh(p, node_hdegs, wmat, bvec)
        lv += 1
        cats.append(cur)

    k1t = K1.T
    k2t = jnp.transpose(K2, (2, 1, 0))
    woutr = jnp.transpose(Wout.reshape(32, 11, 64), (1, 0, 2))
    return _tail(cats[0], cats[1], cats[2], cats[3], k1t, bK1, k2t, bK2,
                 woutr, bout)


# speedup vs baseline: 1.1457x; 1.0661x over previous
import functools

import jax
import jax.numpy as jnp
from jax import lax
from jax.experimental import pallas as pl
from jax.experimental.pallas import tpu as pltpu
from jax.experimental.pallas import tpu_sc as plsc

G = 64
N_PER = 512
N = G * N_PER
M = 8192
NP_ = 8192
MP = 4096
K_SORT = 30

NC = 2
NS = 16
NW = NC * NS
CH = 128
NNZ1 = 524288
NNZ2 = 131072


_SPMEM_BUDGET = 1966080


@functools.lru_cache(maxsize=None)
def _make_spmm(nnz, nrows, d, src_rows, fused=False):
    epw = nnz // NW
    nch = epw // CH
    rpw = nrows // NS
    spw = src_rows // NS
    comb = 3 * CH * d if fused else 0
    sh_extra = 0
    assert not fused or (d % 16 == 0 and spw % CH == 0)
    K = 0
    NB = nch
    for cand in (8, 4, 2):
        if nch % cand == 0 and (
                NS * (cand * CH * d + 2 * nch * CH + comb)
                + nrows * d + sh_extra <= _SPMEM_BUDGET):
            K = cand
            break
    if K < 8 and nch > 16:
        for cand in (8, 4, 3, 2):
            if (NS * (cand * CH * d + 4 * 16 * CH + comb)
                    + nrows * d + sh_extra <= _SPMEM_BUDGET):
                if cand > K:
                    K = cand
                    NB = 16
                break
    assert K >= 2 and nch % NB == 0, (nnz, nrows, d, fused)
    idxw = 2 * nch * CH if NB == nch else 4 * NB * CH
    PIPE = fused and (
        NS * (K * CH * d + idxw + 2 * comb) + nrows * d <= _SPMEM_BUDGET)
    nblk = nch // NB
    NLANE = d // 16
    mesh = plsc.VectorSubcoreMesh(
        core_axis_name="c", subcore_axis_name="s", num_cores=NC,
        num_subcores=NS)

    GA = K // 2
    LG = K - GA

    def body(*refs):
        if fused:
            (x_hbm, rdeg_hbm, gidx_hbm, sidx_hbm, zeros_hbm, out_hbm,
             src2_hbm, gidx_v, sidx_v, rows_v, cb_v, acc_sh, gsem,
             ssem) = refs
        else:
            (x_hbm, gidx_hbm, sidx_hbm, zeros_hbm, out_hbm,
             gidx_v, sidx_v, rows_v, acc_sh, gsem, ssem) = refs
        c = lax.axis_index("c")
        s = lax.axis_index("s")
        w = c * NS + s
        r0 = s * rpw
        pltpu.sync_copy(zeros_hbm.at[pl.ds(r0, rpw), :],
                        acc_sh.at[pl.ds(r0, rpw), :])
        if fused:
            src_sh = src2_hbm.at[c]
            nt = spw // CH

            def pcopies(t):
                tb = (t % 2) if PIPE else 0
                rs = s * spw + t * CH
                return (
                    (x_hbm.at[0, pl.ds(rs, CH), :], cb_v.at[tb, 0]),
                    (x_hbm.at[1, pl.ds(rs, CH), :], cb_v.at[tb, 1]),
                    (rdeg_hbm.at[pl.ds(rs, CH), :], cb_v.at[tb, 2]),
                )

            def pload(t):
                for sr, dst in pcopies(t):
                    pltpu.async_copy(sr, dst, gsem)

            def pwait(t):
                for sr, dst in pcopies(t):
                    pltpu.make_async_copy(sr, dst, gsem).wait()

            pload(0)
            for t in range(nt):
                if PIPE and t + 1 < nt:
                    pload(t + 1)
                pwait(t)
                tb = (t % 2) if PIPE else 0
                rs = s * spw + t * CH

                def vstep(i, carry, tb=tb):
                    for u in range(8):
                        ii = i * 8 + u
                        r = ii // NLANE
                        o = (ii % NLANE) * 16
                        cb_v[tb, 0, r, pl.ds(o, 16)] = (
                            (cb_v[tb, 0, r, pl.ds(o, 16)]
                             + cb_v[tb, 1, r, pl.ds(o, 16)])
                            * cb_v[tb, 2, r, pl.ds(o, 16)])
                    return carry

                lax.fori_loop(0, CH * NLANE // 8, vstep, 0)
                pltpu.sync_copy(cb_v.at[tb, 0], src_sh.at[pl.ds(rs, CH), :])
                if not PIPE and t + 1 < nt:
                    pload(t + 1)
        else:
            src_sh = x_hbm
        base = w * nch

        def stage_idx(b, buf):
            pltpu.sync_copy(gidx_hbm.at[pl.ds(base + b * NB, NB), :],
                            gidx_v.at[buf])
            pltpu.sync_copy(sidx_hbm.at[pl.ds(base + b * NB, NB), :],
                            sidx_v.at[buf])

        stage_idx(0, 0)
        plsc.subcore_barrier()

        def gather(j):
            pltpu.async_copy(
                src_sh.at[gidx_v.at[(j // NB) % 2, j % NB]],
                rows_v.at[pl.ds((j % K) * CH, CH), :], gsem)

        def gather_wait(j):
            pltpu.make_async_copy(
                src_sh.at[gidx_v.at[0, 0]],
                rows_v.at[pl.ds((j % K) * CH, CH), :], gsem).wait()

        def scat(j):
            pltpu.async_copy(
                rows_v.at[pl.ds((j % K) * CH, CH), :],
                acc_sh.at[sidx_v.at[(j // NB) % 2, j % NB]], ssem, add=True)

        def scat_wait(j):
            pltpu.make_async_copy(
                rows_v.at[pl.ds((j % K) * CH, CH), :],
                acc_sh.at[sidx_v.at[0, 0]], ssem).wait()

        for j in range(GA):
            gather(j)

        def blk(b, carry):
            @pl.when(b + 1 < nblk)
            def _():
                stage_idx(b + 1, (b + 1) % 2)

            def step(q, carry2):
                j = b * NB + q

                @pl.when(q >= LG)
                def _():
                    scat_wait(j - LG)

                @pl.when(j + GA < nch)
                def _():
                    gather(j + GA)
                gather_wait(j)
                scat(j)
                return carry2

            lax.fori_loop(0, NB, step, carry)
            def drain(q, carry2):
                scat_wait(b * NB + NB - LG + q)
                return carry2

            lax.fori_loop(0, LG, drain, carry)
            return carry

        lax.fori_loop(0, nblk, blk, 0)
        plsc.subcore_barrier()
        pltpu.sync_copy(acc_sh.at[pl.ds(r0, rpw), :],
                        out_hbm.at[c, pl.ds(r0, rpw), :])

    scratch = [
        pltpu.VMEM((2, NB, CH), jnp.int32) if nblk > 1
        else pltpu.VMEM((1, NB, CH), jnp.int32),
        pltpu.VMEM((2, NB, CH), jnp.int32) if nblk > 1
        else pltpu.VMEM((1, NB, CH), jnp.int32),
        pltpu.VMEM((K * CH, d), jnp.float32),
    ]
    if fused:
        scratch.append(pltpu.VMEM((2 if PIPE else 1, 3, CH, d),
                                  jnp.float32))
    scratch.append(pltpu.VMEM_SHARED((nrows, d), jnp.float32))
    scratch += [pltpu.SemaphoreType.DMA, pltpu.SemaphoreType.DMA]
    out_type = jax.ShapeDtypeStruct((NC, nrows, d), jnp.float32)
    if fused:
        out_type = [out_type,
                    jax.ShapeDtypeStruct((NC, src_rows, d), jnp.float32)]
    return pl.kernel(
        body,
        out_type=out_type,
        mesh=mesh,
        scratch_types=scratch,
        compiler_params=pltpu.CompilerParams(use_tc_tiling_on_sc=False),
        name="sc_spmm%s_%d_%d_%d" % ("f" if fused else "", nnz, nrows, d),
    )


@functools.lru_cache(maxsize=None)
def _make_degs():
    nch1 = (NNZ1 // NW) // CH
    nch2 = (NNZ2 // NW) // CH
    sizes = (N, M, NP_, MP)
    mesh = plsc.VectorSubcoreMesh(
        core_axis_name="c", subcore_axis_name="s", num_cores=NC,
        num_subcores=NS)

    def body(rn_hbm, cn_hbm, rp_hbm, cp_hbm, ones_hbm, zeros_hbm,
             on_hbm, om_hbm, onp_hbm, omp_hbm,
             rn_v, cn_v, rp_v, cp_v, ones_v, an, am, anp, amp, sem):
        c = lax.axis_index("c")
        s = lax.axis_index("s")
        w = c * NS + s
        accs = (an, am, anp, amp)
        outs = (on_hbm, om_hbm, onp_hbm, omp_hbm)
        for acc, r in zip(accs, sizes):
            rp = r // NS
            pltpu.sync_copy(zeros_hbm.at[pl.ds(0, rp), :],
                            acc.at[pl.ds(s * rp, rp), :])
        pltpu.sync_copy(ones_hbm, ones_v)
        pltpu.sync_copy(rn_hbm.at[pl.ds(w * nch1, nch1), :], rn_v)
        pltpu.sync_copy(cn_hbm.at[pl.ds(w * nch1, nch1), :], cn_v)
        pltpu.sync_copy(rp_hbm.at[pl.ds(w * nch2, nch2), :], rp_v)
        pltpu.sync_copy(cp_hbm.at[pl.ds(w * nch2, nch2), :], cp_v)
        plsc.subcore_barrier()

        for idx_v, nchl, acc in ((rn_v, nch1, an), (cn_v, nch1, am),
                                 (rp_v, nch2, anp), (cp_v, nch2, amp)):
            def st(j, carry, idx_v=idx_v, acc=acc):
                pltpu.async_copy(ones_v, acc.at[idx_v.at[j]], sem, add=True)

                @pl.when(j >= 8)
                def _():
                    pltpu.make_async_copy(
                        ones_v, acc.at[idx_v.at[0]], sem).wait()
                return carry

            lax.fori_loop(0, nchl, st, 0)
            for _ in range(min(8, nchl)):
                pltpu.make_async_copy(ones_v, acc.at[idx_v.at[0]],
                                      sem).wait()
        plsc.subcore_barrier()
        for acc, r, out in zip(accs, sizes, outs):
            rp = r // NS
            pltpu.sync_copy(acc.at[pl.ds(s * rp, rp), :],
                            out.at[c, pl.ds(s * rp, rp), :])

    return pl.kernel(
        body,
        out_type=[jax.ShapeDtypeStruct((NC, r, 8), jnp.float32)
                  for r in sizes],
        mesh=mesh,
        scratch_types=[
            pltpu.VMEM((nch1, CH), jnp.int32),
            pltpu.VMEM((nch1, CH), jnp.int32),
            pltpu.VMEM((nch2, CH), jnp.int32),
            pltpu.VMEM((nch2, CH), jnp.int32),
            pltpu.VMEM((CH, 8), jnp.float32),
            pltpu.VMEM_SHARED((N, 8), jnp.float32),
            pltpu.VMEM_SHARED((M, 8), jnp.float32),
            pltpu.VMEM_SHARED((NP_, 8), jnp.float32),
            pltpu.VMEM_SHARED((MP, 8), jnp.float32),
            pltpu.SemaphoreType.DMA,
        ],
        compiler_params=pltpu.CompilerParams(use_tc_tiling_on_sc=False),
        name="sc_degs",
    )


def _spmm(x, gidx2d, sidx2d, nrows):
    nnz = gidx2d.shape[0] * gidx2d.shape[1]
    zeros = jnp.zeros((nrows, x.shape[1]), jnp.float32)
    return _make_spmm(nnz, nrows, x.shape[1], x.shape[0])(
        x, gidx2d, sidx2d, zeros)


def _spmm_f(p, rdeg, gidx2d, sidx2d, nrows):
    nnz = gidx2d.shape[0] * gidx2d.shape[1]
    _, src_rows, d = p.shape
    zeros = jnp.zeros((nrows, d), jnp.float32)
    return _make_spmm(nnz, nrows, d, src_rows, True)(
        p, rdeg, gidx2d, sidx2d, zeros)[0]


def _combine_mul_body(p_ref, rdeg_ref, o_ref):
    o_ref[...] = (p_ref[0] + p_ref[1]) * rdeg_ref[...]


def _combine_mul(p, rdeg):
    _, r, d = p.shape
    br = min(r, 2048)
    return pl.pallas_call(
        _combine_mul_body,
        grid=(r // br,),
        in_specs=[
            pl.BlockSpec((2, br, d), lambda i: (0, i, 0)),
            pl.BlockSpec((br, d), lambda i: (i, 0)),
        ],
        out_specs=pl.BlockSpec((br, d), lambda i: (i, 0)),
        out_shape=jax.ShapeDtypeStruct((r, d), jnp.float32),
    )(p, rdeg)


def _dense_tanh_body(p_ref, deg_ref, w_ref, b_ref, o_ref):
    dout = o_ref.shape[-1]
    pool = p_ref[0] + p_ref[1]
    z = jnp.dot(pool, w_ref[...], preferred_element_type=jnp.float32)
    o_ref[...] = jnp.tanh((z + b_ref[...]) / deg_ref[:, :dout])


def _dense_tanh(p, deg128, wmat, bvec):
    _, r, din = p.shape
    dout = wmat.shape[1]
    br = min(r, 2048)
    return pl.pallas_call(
        _dense_tanh_body,
        grid=(r // br,),
        in_specs=[
            pl.BlockSpec((2, br, din), lambda i: (0, i, 0)),
            pl.BlockSpec((br, 32), lambda i: (i, 0)),
            pl.BlockSpec((din, dout), lambda i: (0, 0)),
            pl.BlockSpec((1, dout), lambda i: (0, 0)),
        ],
        out_specs=pl.BlockSpec((br, dout), lambda i: (i, 0)),
        out_shape=jax.ShapeDtypeStruct((r, dout), jnp.float32),
    )(p, deg128, wmat, bvec.reshape(1, -1))


def _matmul_body(x_ref, w_ref, o_ref):
    o_ref[...] = jnp.dot(x_ref[...], w_ref[...],
                         preferred_element_type=jnp.float32)


def _matmul(x, w):
    r, din = x.shape
    dout = w.shape[1]
    br = min(r, 4096)
    return pl.pallas_call(
        _matmul_body,
        grid=(r // br,),
        in_specs=[
            pl.BlockSpec((br, din), lambda i: (i, 0)),
            pl.BlockSpec((din, dout), lambda i: (0, 0)),
        ],
        out_specs=pl.BlockSpec((br, dout), lambda i: (i, 0)),
        out_shape=jax.ShapeDtypeStruct((r, dout), jnp.float32),
    )(x, w)


def _bexp(width):
    lio = lax.broadcasted_iota(jnp.int32, (128, 16 * width), 0)
    mio = lax.broadcasted_iota(jnp.int32, (128, 16 * width), 1)
    return (lio == 8 * (mio // width)).astype(jnp.float32)


def _deg_expand_body(pn_ref, pm_ref, pnp_ref, pmp_ref,
                     on_ref, onp_ref, rm32_ref, rmp32_ref):
    b32 = _bexp(32)
    for p_ref, o_ref in ((pn_ref, on_ref), (pnp_ref, onp_ref)):
        x = p_ref[0] + p_ref[1]
        o_ref[...] = jnp.dot(x, b32,
                             preferred_element_type=jnp.float32) + 1.0
    for p_ref, o32_ref in ((pm_ref, rm32_ref), (pmp_ref, rmp32_ref)):
        x = p_ref[0] + p_ref[1]
        o32_ref[...] = 1.0 / (jnp.dot(x, b32,
                                      preferred_element_type=jnp.float32)
                              + 1.0)


def _finalize_degs(pn, pm, pnp, pmp):
    outs = pl.pallas_call(
        _deg_expand_body,
        out_shape=[
            jax.ShapeDtypeStruct((N // 16, 512), jnp.float32),
            jax.ShapeDtypeStruct((NP_ // 16, 512), jnp.float32),
            jax.ShapeDtypeStruct((M // 16, 512), jnp.float32),
            jax.ShapeDtypeStruct((MP // 16, 512), jnp.float32),
        ],
    )(*[p.reshape(2, p.shape[1] // 16, 128) for p in (pn, pm, pnp, pmp)])
    shp = ((N, 32), (NP_, 32), (M, 32), (MP, 32))
    return [o.reshape(s) for o, s in zip(outs, shp)]


def _tail_body(c0_ref, c1_ref, c2_ref, c3c_ref, c3r_ref, k1t_ref, bk1_ref,
               k2t_ref, bk2_ref, wout_ref, bout_ref, o_ref):
    vrow = c3r_ref[...].reshape(1, N_PER)
    vcol = c3c_ref[...]
    jp = lax.broadcasted_iota(jnp.int32, (N_PER, N_PER), 1)
    jj = lax.broadcasted_iota(jnp.int32, (N_PER, N_PER), 0)
    ahead = (vrow > vcol) | ((vrow == vcol) & (jp < jj))
    rank = jnp.sum(ahead.astype(jnp.float32), axis=1, keepdims=True)
    kio = lax.broadcasted_iota(jnp.int32, (N_PER, K_SORT), 1).astype(
        jnp.float32)
    sel = (rank == kio).astype(jnp.float32)

    def pool_t(x):
        return lax.dot_general(sel, x, (((0,), (0,)), ((), ())),
                               preferred_element_type=jnp.float32)

    z1 = (jnp.dot(pool_t(c0_ref[...]), k1t_ref[0:32, :],
                  preferred_element_type=jnp.float32)
          + jnp.dot(pool_t(c1_ref[...]), k1t_ref[32:64, :],
                    preferred_element_type=jnp.float32)
          + jnp.dot(pool_t(c2_ref[...]), k1t_ref[64:96, :],
                    preferred_element_type=jnp.float32)
          + jnp.dot(pool_t(vcol), k1t_ref[96:97, :],
                    preferred_element_type=jnp.float32))
    z1 = jnp.maximum(z1 + bk1_ref[...], 0.0)
    wio = lax.broadcasted_iota(jnp.int32, (K_SORT, K_SORT // 2), 1)
    jio = lax.broadcasted_iota(jnp.int32, (K_SORT, K_SORT // 2), 0)
    s_even = (jio == 2 * wio).astype(jnp.float32)
    s_odd = (jio == 2 * wio + 1).astype(jnp.float32)

    def sel_t(smat):
        return lax.dot_general(smat, z1, (((0,), (0,)), ((), ())),
                               preferred_element_type=jnp.float32)

    zp = jnp.maximum(sel_t(s_even), sel_t(s_odd))
    c2 = jnp.zeros((11, 32), jnp.float32)
    for t in range(5):
        c2 = c2 + jnp.dot(zp[t:t + 11, :], k2t_ref[t],
                          preferred_element_type=jnp.float32)
    c2 = jnp.maximum(c2 + bk2_ref[...], 0.0)
    acc = jnp.zeros((1, 64), jnp.float32)
    for w in range(11):
        acc = acc + jnp.dot(c2[w:w + 1, :], wout_ref[w],
                            preferred_element_type=jnp.float32)
    o_ref[...] = jnp.maximum(acc + bout_ref[...], 0.0).reshape(1, 1, 64)


def _tail(c0, c1, c2, c3, k1t, bk1, k2t, bk2, woutr, bout):
    c3row = c3.reshape(G, 1, N_PER)
    grid = (G,)
    out = pl.pallas_call(
        _tail_body,
        grid=grid,
        in_specs=[
            pl.BlockSpec((N_PER, 32), lambda g: (g, 0)),
            pl.BlockSpec((N_PER, 32), lambda g: (g, 0)),
            pl.BlockSpec((N_PER, 32), lambda g: (g, 0)),
            pl.BlockSpec((N_PER, 1), lambda g: (g, 0)),
            pl.BlockSpec((1, 1, N_PER), lambda g: (g, 0, 0)),
            pl.BlockSpec((97, 16), lambda g: (0, 0)),
            pl.BlockSpec((1, 16), lambda g: (0, 0)),
            pl.BlockSpec((5, 16, 32), lambda g: (0, 0, 0)),
            pl.BlockSpec((1, 32), lambda g: (0, 0)),
            pl.BlockSpec((11, 32, 64), lambda g: (0, 0, 0)),
            pl.BlockSpec((1, 64), lambda g: (0, 0)),
        ],
        out_specs=pl.BlockSpec((1, 1, 64), lambda g: (g, 0, 0)),
        out_shape=jax.ShapeDtypeStruct((G, 1, 64), jnp.float32),
    )(c0, c1, c2, c3, c3row, k1t, bk1.reshape(1, 16), k2t,
      bk2.reshape(1, 32), woutr, bout.reshape(1, 64))
    return out.reshape(G, 64)


def kernel(node_feat, n2m_row, n2m_col, np2mp_row, np2mp_col, m2mp_row,
           m2mp_col, W0, b0, W1, b1, W2, b2, W3, b3, W4, b4, W5, b5, W6, b6,
           W7, b7, K1, bK1, K2, bK2, Wout, bout):
    r_n2m = n2m_row.reshape(-1, CH)
    c_n2m = n2m_col.reshape(-1, CH)
    r_np2mp = np2mp_row.reshape(-1, CH)
    c_np2mp = np2mp_col.reshape(-1, CH)
    r_m2mp = m2mp_row.reshape(-1, CH)
    c_m2mp = m2mp_col.reshape(-1, CH)

    ones8 = jnp.ones((CH, 8), jnp.float32)
    zeros8 = jnp.zeros((N // NS, 8), jnp.float32)
    pn, pm, pnp, pmp = _make_degs()(r_n2m, c_n2m, r_np2mp, c_np2mp,
                                    ones8, zeros8)
    node_hdegs, node_hdegs_, rM32, rMP32 = _finalize_degs(pn, pm, pnp, pmp)

    Ws = [(jnp.eye(32, dtype=jnp.float32), b0), (W1, b1), (W2, b2),
          (W3, b3), (W4, b4), (W5, b5),
          (jnp.pad(W6, ((0, 0), (0, 31))), jnp.pad(b6, (0, 31))),
          (jnp.pad(W7, ((0, 31), (0, 0))), b7)]

    cur = _matmul(node_feat, W0)
    cats = []
    lv = 0
    for it in range(4):
        p = _spmm(cur, r_n2m, c_n2m, M)
        p = _spmm_f(p, rM32, r_m2mp, c_m2mp, MP)
        p = _spmm_f(p, rMP32, c_np2mp, r_np2mp, NP_)
        wmat, bvec = Ws[lv]
        cur_ = _dense_tanh(p, node_hdegs_, wmat, bvec)
        lv += 1
        p = _spmm(cur_, r_np2mp, c_np2mp, MP)
        p = _spmm_f(p, rMP32, c_m2mp, r_m2mp, M)
        p = _spmm_f(p, rM32, c_n2m, r_n2m, N)
        wmat, bvec = Ws[lv]
        cur = _dense_tanh(p, node_hdegs, wmat, bvec)
        lv += 1
        cats.append(cur)

    k1t = K1.T
    k2t = jnp.transpose(K2, (2, 1, 0))
    woutr = jnp.transpose(Wout.reshape(32, 11, 64), (1, 0, 2))
    return _tail(cats[0], cats[1], cats[2], cats[3], k1t, bK1, k2t, bK2,
                 woutr, bout)


# speedup vs baseline: 1.1459x; 1.0002x over previous
import functools

import jax
import jax.numpy as jnp
from jax import lax
from jax.experimental import pallas as pl
from jax.experimental.pallas import tpu as pltpu
from jax.experimental.pallas import tpu_sc as plsc

G = 64
N_PER = 512
N = G * N_PER
M = 8192
NP_ = 8192
MP = 4096
K_SORT = 30

NC = 2
NS = 16
NW = NC * NS
CH = 128
NNZ1 = 524288
NNZ2 = 131072


_SPMEM_BUDGET = 1966080


@functools.lru_cache(maxsize=None)
def _make_spmm(nnz, nrows, d, src_rows, fused=False):
    epw = nnz // NW
    nch = epw // CH
    rpw = nrows // NS
    spw = src_rows // NS
    comb = 3 * CH * d if fused else 0
    sh_extra = 0
    assert not fused or (d % 16 == 0 and spw % CH == 0)
    K = 0
    NB = nch
    for cand in (8, 4, 2):
        if nch % cand == 0 and (
                NS * (cand * CH * d + 2 * nch * CH + comb)
                + nrows * d + sh_extra <= _SPMEM_BUDGET):
            K = cand
            break
    if K < 8 and nch > 16:
        for cand in (8, 4, 3, 2):
            if (NS * (cand * CH * d + 4 * 16 * CH + comb)
                    + nrows * d + sh_extra <= _SPMEM_BUDGET):
                if cand > K:
                    K = cand
                    NB = 16
                break
    assert K >= 2 and nch % NB == 0, (nnz, nrows, d, fused)
    idxw = 2 * nch * CH if NB == nch else 4 * NB * CH
    PIPE = fused and (
        NS * (K * CH * d + idxw + 2 * comb) + nrows * d <= _SPMEM_BUDGET)
    nblk = nch // NB
    NLANE = d // 16
    mesh = plsc.VectorSubcoreMesh(
        core_axis_name="c", subcore_axis_name="s", num_cores=NC,
        num_subcores=NS)

    GA = K // 2
    LG = K - GA

    def body(*refs):
        if fused:
            (x_hbm, rdeg_hbm, gidx_hbm, sidx_hbm, zeros_hbm, out_hbm,
             src2_hbm, gidx_v, sidx_v, rows_v, cb_v, acc_sh, gsem,
             ssem) = refs
        else:
            (x_hbm, gidx_hbm, sidx_hbm, zeros_hbm, out_hbm,
             gidx_v, sidx_v, rows_v, acc_sh, gsem, ssem) = refs
        c = lax.axis_index("c")
        s = lax.axis_index("s")
        w = c * NS + s
        r0 = s * rpw
        pltpu.sync_copy(zeros_hbm.at[pl.ds(r0, rpw), :],
                        acc_sh.at[pl.ds(r0, rpw), :])
        if fused:
            src_sh = src2_hbm.at[c]
            nt = spw // CH

            def pcopies(t):
                tb = (t % 2) if PIPE else 0
                rs = s * spw + t * CH
                return (
                    (x_hbm.at[0, pl.ds(rs, CH), :], cb_v.at[tb, 0]),
                    (x_hbm.at[1, pl.ds(rs, CH), :], cb_v.at[tb, 1]),
                    (rdeg_hbm.at[pl.ds(rs, CH), :], cb_v.at[tb, 2]),
                )

            def pload(t):
                for sr, dst in pcopies(t):
                    pltpu.async_copy(sr, dst, gsem)

            def pwait(t):
                for sr, dst in pcopies(t):
                    pltpu.make_async_copy(sr, dst, gsem).wait()

            pload(0)
            for t in range(nt):
                if PIPE and t + 1 < nt:
                    pload(t + 1)
                pwait(t)
                tb = (t % 2) if PIPE else 0
                rs = s * spw + t * CH

                def vstep(i, carry, tb=tb):
                    for u in range(8):
                        ii = i * 8 + u
                        r = ii // NLANE
                        o = (ii % NLANE) * 16
                        cb_v[tb, 0, r, pl.ds(o, 16)] = (
                            (cb_v[tb, 0, r, pl.ds(o, 16)]
                             + cb_v[tb, 1, r, pl.ds(o, 16)])
                            * cb_v[tb, 2, r, pl.ds(o, 16)])
                    return carry

                lax.fori_loop(0, CH * NLANE // 8, vstep, 0)
                pltpu.sync_copy(cb_v.at[tb, 0], src_sh.at[pl.ds(rs, CH), :])
                if not PIPE and t + 1 < nt:
                    pload(t + 1)
        else:
            src_sh = x_hbm
        base = w * nch

        def stage_idx(b, buf):
            pltpu.sync_copy(gidx_hbm.at[pl.ds(base + b * NB, NB), :],
                            gidx_v.at[buf])
            pltpu.sync_copy(sidx_hbm.at[pl.ds(base + b * NB, NB), :],
                            sidx_v.at[buf])

        stage_idx(0, 0)
        plsc.subcore_barrier()

        def gather(j):
            pltpu.async_copy(
                src_sh.at[gidx_v.at[(j // NB) % 2, j % NB]],
                rows_v.at[pl.ds((j % K) * CH, CH), :], gsem)

        def gather_wait(j):
            pltpu.make_async_copy(
                src_sh.at[gidx_v.at[0, 0]],
                rows_v.at[pl.ds((j % K) * CH, CH), :], gsem).wait()

        def scat(j):
            pltpu.async_copy(
                rows_v.at[pl.ds((j % K) * CH, CH), :],
                acc_sh.at[sidx_v.at[(j // NB) % 2, j % NB]], ssem, add=True)

        def scat_wait(j):
            pltpu.make_async_copy(
                rows_v.at[pl.ds((j % K) * CH, CH), :],
                acc_sh.at[sidx_v.at[0, 0]], ssem).wait()

        for j in range(GA):
            gather(j)

        def blk(b, carry):
            @pl.when(b + 1 < nblk)
            def _():
                stage_idx(b + 1, (b + 1) % 2)

            def step(q, carry2):
                j = b * NB + q

                @pl.when(q >= LG)
                def _():
                    scat_wait(j - LG)

                @pl.when(j + GA < nch)
                def _():
                    gather(j + GA)
                gather_wait(j)
                scat(j)
                return carry2

            lax.fori_loop(0, NB, step, carry)
            def drain(q, carry2):
                scat_wait(b * NB + NB - LG + q)
                return carry2

            lax.fori_loop(0, LG, drain, carry)
            return carry

        lax.fori_loop(0, nblk, blk, 0)
        plsc.subcore_barrier()
        pltpu.sync_copy(acc_sh.at[pl.ds(r0, rpw), :],
                        out_hbm.at[c, pl.ds(r0, rpw), :])

    scratch = [
        pltpu.VMEM((2, NB, CH), jnp.int32) if nblk > 1
        else pltpu.VMEM((1, NB, CH), jnp.int32),
        pltpu.VMEM((2, NB, CH), jnp.int32) if nblk > 1
        else pltpu.VMEM((1, NB, CH), jnp.int32),
        pltpu.VMEM((K * CH, d), jnp.float32),
    ]
    if fused:
        scratch.append(pltpu.VMEM((2 if PIPE else 1, 3, CH, d),
                                  jnp.float32))
    scratch.append(pltpu.VMEM_SHARED((nrows, d), jnp.float32))
    scratch += [pltpu.SemaphoreType.DMA, pltpu.SemaphoreType.DMA]
    out_type = jax.ShapeDtypeStruct((NC, nrows, d), jnp.float32)
    if fused:
        out_type = [out_type,
                    jax.ShapeDtypeStruct((NC, src_rows, d), jnp.float32)]
    return pl.kernel(
        body,
        out_type=out_type,
        mesh=mesh,
        scratch_types=scratch,
        compiler_params=pltpu.CompilerParams(use_tc_tiling_on_sc=False),
        name="sc_spmm%s_%d_%d_%d" % ("f" if fused else "", nnz, nrows, d),
    )


@functools.lru_cache(maxsize=None)
def _make_degs():
    nch1 = (NNZ1 // NW) // CH
    nch2 = (NNZ2 // NW) // CH
    sizes = (N, M, NP_, MP)
    mesh = plsc.VectorSubcoreMesh(
        core_axis_name="c", subcore_axis_name="s", num_cores=NC,
        num_subcores=NS)

    def body(rn_hbm, cn_hbm, rp_hbm, cp_hbm, ones_hbm, zeros_hbm,
             on_hbm, om_hbm, onp_hbm, omp_hbm,
             rn_v, cn_v, rp_v, cp_v, ones_v, an, am, anp, amp, sem):
        c = lax.axis_index("c")
        s = lax.axis_index("s")
        w = c * NS + s
        accs = (an, am, anp, amp)
        outs = (on_hbm, om_hbm, onp_hbm, omp_hbm)
        for acc, r in zip(accs, sizes):
            rp = r // NS
            pltpu.sync_copy(zeros_hbm.at[pl.ds(0, rp), :],
                            acc.at[pl.ds(s * rp, rp), :])
        pltpu.sync_copy(ones_hbm, ones_v)
        pltpu.sync_copy(rn_hbm.at[pl.ds(w * nch1, nch1), :], rn_v)
        pltpu.sync_copy(cn_hbm.at[pl.ds(w * nch1, nch1), :], cn_v)
        pltpu.sync_copy(rp_hbm.at[pl.ds(w * nch2, nch2), :], rp_v)
        pltpu.sync_copy(cp_hbm.at[pl.ds(w * nch2, nch2), :], cp_v)
        plsc.subcore_barrier()

        for idx_v, nchl, acc in ((rn_v, nch1, an), (cn_v, nch1, am),
                                 (rp_v, nch2, anp), (cp_v, nch2, amp)):
            def st(j, carry, idx_v=idx_v, acc=acc):
                pltpu.async_copy(ones_v, acc.at[idx_v.at[j]], sem, add=True)

                @pl.when(j >= 8)
                def _():
                    pltpu.make_async_copy(
                        ones_v, acc.at[idx_v.at[0]], sem).wait()
                return carry

            lax.fori_loop(0, nchl, st, 0)
            for _ in range(min(8, nchl)):
                pltpu.make_async_copy(ones_v, acc.at[idx_v.at[0]],
                                      sem).wait()
        plsc.subcore_barrier()
        for acc, r, out in zip(accs, sizes, outs):
            rp = r // NS
            pltpu.sync_copy(acc.at[pl.ds(s * rp, rp), :],
                            out.at[c, pl.ds(s * rp, rp), :])

    return pl.kernel(
        body,
        out_type=[jax.ShapeDtypeStruct((NC, r, 8), jnp.float32)
                  for r in sizes],
        mesh=mesh,
        scratch_types=[
            pltpu.VMEM((nch1, CH), jnp.int32),
            pltpu.VMEM((nch1, CH), jnp.int32),
            pltpu.VMEM((nch2, CH), jnp.int32),
            pltpu.VMEM((nch2, CH), jnp.int32),
            pltpu.VMEM((CH, 8), jnp.float32),
            pltpu.VMEM_SHARED((N, 8), jnp.float32),
            pltpu.VMEM_SHARED((M, 8), jnp.float32),
            pltpu.VMEM_SHARED((NP_, 8), jnp.float32),
            pltpu.VMEM_SHARED((MP, 8), jnp.float32),
            pltpu.SemaphoreType.DMA,
        ],
        compiler_params=pltpu.CompilerParams(use_tc_tiling_on_sc=False),
        name="sc_degs",
    )


def _spmm(x, gidx2d, sidx2d, nrows):
    nnz = gidx2d.shape[0] * gidx2d.shape[1]
    zeros = jnp.zeros((nrows, x.shape[1]), jnp.float32)
    return _make_spmm(nnz, nrows, x.shape[1], x.shape[0])(
        x, gidx2d, sidx2d, zeros)


def _spmm_f(p, rdeg, gidx2d, sidx2d, nrows):
    nnz = gidx2d.shape[0] * gidx2d.shape[1]
    _, src_rows, d = p.shape
    zeros = jnp.zeros((nrows, d), jnp.float32)
    return _make_spmm(nnz, nrows, d, src_rows, True)(
        p, rdeg, gidx2d, sidx2d, zeros)[0]


def _dense_tanh_body(p_ref, deg_ref, w_ref, b_ref, o_ref):
    dout = o_ref.shape[-1]
    pool = p_ref[0] + p_ref[1]
    z = jnp.dot(pool, w_ref[...], preferred_element_type=jnp.float32)
    o_ref[...] = jnp.tanh((z + b_ref[...]) / deg_ref[:, :dout])


def _dense_tanh(p, deg128, wmat, bvec):
    _, r, din = p.shape
    dout = wmat.shape[1]
    br = min(r, 2048)
    return pl.pallas_call(
        _dense_tanh_body,
        grid=(r // br,),
        in_specs=[
            pl.BlockSpec((2, br, din), lambda i: (0, i, 0)),
            pl.BlockSpec((br, 32), lambda i: (i, 0)),
            pl.BlockSpec((din, dout), lambda i: (0, 0)),
            pl.BlockSpec((1, dout), lambda i: (0, 0)),
        ],
        out_specs=pl.BlockSpec((br, dout), lambda i: (i, 0)),
        out_shape=jax.ShapeDtypeStruct((r, dout), jnp.float32),
    )(p, deg128, wmat, bvec.reshape(1, -1))


def _matmul_body(x_ref, w_ref, o_ref):
    o_ref[...] = jnp.dot(x_ref[...], w_ref[...],
                         preferred_element_type=jnp.float32)


def _matmul(x, w):
    r, din = x.shape
    dout = w.shape[1]
    br = min(r, 4096)
    return pl.pallas_call(
        _matmul_body,
        grid=(r // br,),
        in_specs=[
            pl.BlockSpec((br, din), lambda i: (i, 0)),
            pl.BlockSpec((din, dout), lambda i: (0, 0)),
        ],
        out_specs=pl.BlockSpec((br, dout), lambda i: (i, 0)),
        out_shape=jax.ShapeDtypeStruct((r, dout), jnp.float32),
    )(x, w)


def _bexp(width):
    lio = lax.broadcasted_iota(jnp.int32, (128, 16 * width), 0)
    mio = lax.broadcasted_iota(jnp.int32, (128, 16 * width), 1)
    return (lio == 8 * (mio // width)).astype(jnp.float32)


def _deg_expand_body(pn_ref, pm_ref, pnp_ref, pmp_ref,
                     on_ref, onp_ref, rm32_ref, rmp32_ref):
    b32 = _bexp(32)
    for p_ref, o_ref in ((pn_ref, on_ref), (pnp_ref, onp_ref)):
        x = p_ref[0] + p_ref[1]
        o_ref[...] = jnp.dot(x, b32,
                             preferred_element_type=jnp.float32) + 1.0
    for p_ref, o32_ref in ((pm_ref, rm32_ref), (pmp_ref, rmp32_ref)):
        x = p_ref[0] + p_ref[1]
        o32_ref[...] = 1.0 / (jnp.dot(x, b32,
                                      preferred_element_type=jnp.float32)
                              + 1.0)


def _finalize_degs(pn, pm, pnp, pmp):
    outs = pl.pallas_call(
        _deg_expand_body,
        out_shape=[
            jax.ShapeDtypeStruct((N // 16, 512), jnp.float32),
            jax.ShapeDtypeStruct((NP_ // 16, 512), jnp.float32),
            jax.ShapeDtypeStruct((M // 16, 512), jnp.float32),
            jax.ShapeDtypeStruct((MP // 16, 512), jnp.float32),
        ],
    )(*[p.reshape(2, p.shape[1] // 16, 128) for p in (pn, pm, pnp, pmp)])
    shp = ((N, 32), (NP_, 32), (M, 32), (MP, 32))
    return [o.reshape(s) for o, s in zip(outs, shp)]


def _tail_body(c0_ref, c1_ref, c2_ref, c3c_ref, c3r_ref, k1t_ref, bk1_ref,
               k2t_ref, bk2_ref, wout_ref, bout_ref, o_ref):
    vrow = c3r_ref[...].reshape(1, N_PER)
    vcol = c3c_ref[...]
    jp = lax.broadcasted_iota(jnp.int32, (N_PER, N_PER), 1)
    jj = lax.broadcasted_iota(jnp.int32, (N_PER, N_PER), 0)
    ahead = (vrow > vcol) | ((vrow == vcol) & (jp < jj))
    rank = jnp.sum(ahead.astype(jnp.float32), axis=1, keepdims=True)
    kio = lax.broadcasted_iota(jnp.int32, (N_PER, K_SORT), 1).astype(
        jnp.float32)
    sel = (rank == kio).astype(jnp.float32)

    def pool_t(x):
        return lax.dot_general(sel, x, (((0,), (0,)), ((), ())),
                               preferred_element_type=jnp.float32)

    z1 = (jnp.dot(pool_t(c0_ref[...]), k1t_ref[0:32, :],
                  preferred_element_type=jnp.float32)
          + jnp.dot(pool_t(c1_ref[...]), k1t_ref[32:64, :],
                    preferred_element_type=jnp.float32)
          + jnp.dot(pool_t(c2_ref[...]), k1t_ref[64:96, :],
                    preferred_element_type=jnp.float32)
          + jnp.dot(pool_t(vcol), k1t_ref[96:97, :],
                    preferred_element_type=jnp.float32))
    z1 = jnp.maximum(z1 + bk1_ref[...], 0.0)
    wio = lax.broadcasted_iota(jnp.int32, (K_SORT, K_SORT // 2), 1)
    jio = lax.broadcasted_iota(jnp.int32, (K_SORT, K_SORT // 2), 0)
    s_even = (jio == 2 * wio).astype(jnp.float32)
    s_odd = (jio == 2 * wio + 1).astype(jnp.float32)

    def sel_t(smat):
        return lax.dot_general(smat, z1, (((0,), (0,)), ((), ())),
                               preferred_element_type=jnp.float32)

    zp = jnp.maximum(sel_t(s_even), sel_t(s_odd))
    c2 = jnp.zeros((11, 32), jnp.float32)
    for t in range(5):
        c2 = c2 + jnp.dot(zp[t:t + 11, :], k2t_ref[t],
                          preferred_element_type=jnp.float32)
    c2 = jnp.maximum(c2 + bk2_ref[...], 0.0)
    acc = jnp.zeros((1, 64), jnp.float32)
    for w in range(11):
        acc = acc + jnp.dot(c2[w:w + 1, :], wout_ref[w],
                            preferred_element_type=jnp.float32)
    o_ref[...] = jnp.maximum(acc + bout_ref[...], 0.0).reshape(1, 1, 64)


def _tail(c0, c1, c2, c3, k1t, bk1, k2t, bk2, woutr, bout):
    c3row = c3.reshape(G, 1, N_PER)
    grid = (G,)
    out = pl.pallas_call(
        _tail_body,
        grid=grid,
        in_specs=[
            pl.BlockSpec((N_PER, 32), lambda g: (g, 0)),
            pl.BlockSpec((N_PER, 32), lambda g: (g, 0)),
            pl.BlockSpec((N_PER, 32), lambda g: (g, 0)),
            pl.BlockSpec((N_PER, 1), lambda g: (g, 0)),
            pl.BlockSpec((1, 1, N_PER), lambda g: (g, 0, 0)),
            pl.BlockSpec((97, 16), lambda g: (0, 0)),
            pl.BlockSpec((1, 16), lambda g: (0, 0)),
            pl.BlockSpec((5, 16, 32), lambda g: (0, 0, 0)),
            pl.BlockSpec((1, 32), lambda g: (0, 0)),
            pl.BlockSpec((11, 32, 64), lambda g: (0, 0, 0)),
            pl.BlockSpec((1, 64), lambda g: (0, 0)),
        ],
        out_specs=pl.BlockSpec((1, 1, 64), lambda g: (g, 0, 0)),
        out_shape=jax.ShapeDtypeStruct((G, 1, 64), jnp.float32),
    )(c0, c1, c2, c3, c3row, k1t, bk1.reshape(1, 16), k2t,
      bk2.reshape(1, 32), woutr, bout.reshape(1, 64))
    return out.reshape(G, 64)


def kernel(node_feat, n2m_row, n2m_col, np2mp_row, np2mp_col, m2mp_row,
           m2mp_col, W0, b0, W1, b1, W2, b2, W3, b3, W4, b4, W5, b5, W6, b6,
           W7, b7, K1, bK1, K2, bK2, Wout, bout):
    r_n2m = n2m_row.reshape(-1, CH)
    c_n2m = n2m_col.reshape(-1, CH)
    r_np2mp = np2mp_row.reshape(-1, CH)
    c_np2mp = np2mp_col.reshape(-1, CH)
    r_m2mp = m2mp_row.reshape(-1, CH)
    c_m2mp = m2mp_col.reshape(-1, CH)

    ones8 = jnp.ones((CH, 8), jnp.float32)
    zeros8 = jnp.zeros((N // NS, 8), jnp.float32)
    pn, pm, pnp, pmp = _make_degs()(r_n2m, c_n2m, r_np2mp, c_np2mp,
                                    ones8, zeros8)
    node_hdegs, node_hdegs_, rM32, rMP32 = _finalize_degs(pn, pm, pnp, pmp)

    Ws = [(jnp.eye(32, dtype=jnp.float32), b0), (W1, b1), (W2, b2),
          (W3, b3), (W4, b4), (W5, b5),
          (jnp.pad(W6, ((0, 0), (0, 31))), jnp.pad(b6, (0, 31))),
          (jnp.pad(W7, ((0, 31), (0, 0))), b7)]

    cur = _matmul(node_feat, W0)
    cats = []
    lv = 0
    for it in range(4):
        p = _spmm(cur, r_n2m, c_n2m, M)
        p = _spmm_f(p, rM32, r_m2mp, c_m2mp, MP)
        p = _spmm_f(p, rMP32, c_np2mp, r_np2mp, NP_)
        wmat, bvec = Ws[lv]
        cur_ = _dense_tanh(p, node_hdegs_, wmat, bvec)
        lv += 1
        p = _spmm(cur_, r_np2mp, c_np2mp, MP)
        p = _spmm_f(p, rMP32, c_m2mp, r_m2mp, M)
        p = _spmm_f(p, rM32, c_n2m, r_n2m, N)
        wmat, bvec = Ws[lv]
        cur = _dense_tanh(p, node_hdegs, wmat, bvec)
        lv += 1
        cats.append(cur)

    k1t = K1.T
    k2t = jnp.transpose(K2, (2, 1, 0))
    woutr = jnp.transpose(Wout.reshape(32, 11, 64), (1, 0, 2))
    return _tail(cats[0], cats[1], cats[2], cats[3], k1t, bK1, k2t, bK2,
                 woutr, bout)
